# Initial kernel scaffold; baseline (speedup 1.0000x reference)
import jax
import jax.numpy as jnp
from jax.experimental import pallas as pl


def kernel(x, edge_index, W1, a_src1, a_dst1, b1, W2, a_src2, a_dst2, b2):
    raise NotImplementedError("write your pallas kernel here")


# speedup vs baseline: 1.1512x; 1.1512x over previous
import functools

import jax
import jax.numpy as jnp
from jax.experimental import pallas as pl
from jax.experimental.pallas import tpu as pltpu

N = 10000
E = 320000
D_IN = 128
HID = 64
HEADS = 8
D_OUT = 128

_BLK = 2000


def _mm_body(x_ref, w_ref, o_ref):
    o_ref[...] = jnp.dot(x_ref[...], w_ref[...], preferred_element_type=jnp.float32)


def _matmul(x, w):
    n, k = x.shape
    _, m = w.shape
    grid = (n // _BLK,)
    return pl.pallas_call(
        _mm_body,
        grid=grid,
        in_specs=[
            pl.BlockSpec((_BLK, k), lambda i: (i, 0)),
            pl.BlockSpec((k, m), lambda i: (0, 0)),
        ],
        out_specs=pl.BlockSpec((_BLK, m), lambda i: (i, 0)),
        out_shape=jax.ShapeDtypeStruct((n, m), jnp.float32),
    )(x, w)


def _gat_conv(x, edge_index, W, a_s, a_d, b, heads, ch, concat):
    n = x.shape[0]
    loop = jnp.arange(n, dtype=edge_index.dtype)
    src = jnp.concatenate([edge_index[0], loop])
    dst = jnp.concatenate([edge_index[1], loop])
    h = _matmul(x, W).reshape(n, heads, ch)
    alpha_s = (h * a_s[None, :, :]).sum(-1)
    alpha_d = (h * a_d[None, :, :]).sum(-1)
    e = alpha_s[src] + alpha_d[dst]
    e = jax.nn.leaky_relu(e, 0.2)
    p = jnp.exp(e)
    s = jax.ops.segment_sum(p, dst, num_segments=n)
    out = jax.ops.segment_sum(h[src] * p[:, :, None], dst, num_segments=n)
    out = out / (s[:, :, None] + 1e-16)
    out = out.reshape(n, heads * ch) if concat else out.mean(axis=1)
    return out + b


def kernel(x, edge_index, W1, a_src1, a_dst1, b1, W2, a_src2, a_dst2, b2):
    h = _gat_conv(x, edge_index, W1, a_src1, a_dst1, b1, HEADS, HID, True)
    h = jax.nn.relu(h)
    h = _gat_conv(h, edge_index, W2, a_src2, a_dst2, b2, 1, D_OUT, False)
    return jax.nn.log_softmax(h, axis=1)


# speedup vs baseline: 16.7852x; 14.5807x over previous
import functools

import jax
import jax.numpy as jnp
from jax import lax
from jax.experimental import pallas as pl
from jax.experimental.pallas import tpu as pltpu
from jax.experimental.pallas import tpu_sc as plsc

N = 10000
E = 320000
D_IN = 128
HID = 64
HEADS = 8
D_OUT = 128

NC = 2
NS = 16
NW = NC * NS
EPW = E // NW
CH = 80
NCHUNK = EPW // CH
NPAD = 10240
RPT = NPAD // NS

_BLK = 2000

_f32 = jnp.float32
_i32 = jnp.int32


def _stage1_body(x_ref, w1_ref, as_ref, ad_ref,
                 ht0, ht1, ht2, ht3, p1tab, ploop):
    h = jnp.dot(x_ref[...], w1_ref[...], preferred_element_type=_f32)
    ht0[...] = h[:, 0:128]
    ht1[...] = h[:, 128:256]
    ht2[...] = h[:, 256:384]
    ht3[...] = h[:, 384:512]
    a_s = jnp.dot(h, as_ref[...], preferred_element_type=_f32)
    a_d = jnp.dot(h, ad_ref[...], preferred_element_type=_f32)
    p1tab[...] = jnp.concatenate([a_s, a_d], axis=1)
    el = a_s + a_d
    el = jnp.where(el >= 0, el, 0.2 * el)
    ploop[...] = jnp.exp(el)


def _stage1(x, W1, As1, Ad1):
    grid = (N // _BLK,)
    return pl.pallas_call(
        _stage1_body,
        grid=grid,
        in_specs=[
            pl.BlockSpec((_BLK, D_IN), lambda i: (i, 0)),
            pl.BlockSpec((D_IN, HEADS * HID), lambda i: (0, 0)),
            pl.BlockSpec((HEADS * HID, HEADS), lambda i: (0, 0)),
            pl.BlockSpec((HEADS * HID, HEADS), lambda i: (0, 0)),
        ],
        out_specs=[
            pl.BlockSpec((_BLK, 128), lambda i: (i, 0)),
            pl.BlockSpec((_BLK, 128), lambda i: (i, 0)),
            pl.BlockSpec((_BLK, 128), lambda i: (i, 0)),
            pl.BlockSpec((_BLK, 128), lambda i: (i, 0)),
            pl.BlockSpec((_BLK, 16), lambda i: (i, 0)),
            pl.BlockSpec((_BLK, 8), lambda i: (i, 0)),
        ],
        out_shape=[
            jax.ShapeDtypeStruct((N, 128), _f32),
            jax.ShapeDtypeStruct((N, 128), _f32),
            jax.ShapeDtypeStruct((N, 128), _f32),
            jax.ShapeDtypeStruct((N, 128), _f32),
            jax.ShapeDtypeStruct((N, 16), _f32),
            jax.ShapeDtypeStruct((N, 8), _f32),
        ],
    )(x, W1, As1, Ad1)


def _sc1_body(src_hbm, dst_hbm, p1tab_hbm, ht0, ht1, ht2, ht3,
              p1_out, s1_out, acc1_out,
              s_sh, acc_sh,
              src_v, dst_v, srows, drows, pch, pt, rows, pav, pbv,
              zb16, zb128, sem):
    cid = lax.axis_index("c")
    sid = lax.axis_index("s")
    wid = sid * NC + cid
    ebase = wid * EPW
    rbase = sid * RPT

    zv = jnp.zeros((16,), _f32)
    iota16 = lax.iota(_i32, 16)

    def _zb_init(i, c):
        zb16[i, :] = zv
        for j in range(8):
            zb128[i, pl.ds(j * 16, 16)] = zv
        return c
    lax.fori_loop(0, 128, _zb_init, 0)

    def _pt_init(i, c):
        pt[i, :] = zv
        return c
    lax.fori_loop(0, CH, _pt_init, 0)

    for k in range(5):
        pltpu.sync_copy(zb16, s_sh.at[pl.ds(rbase + k * 128, 128)])
    plsc.subcore_barrier()

    def _chunk_a(ci, c):
        off = ebase + ci * CH
        pltpu.sync_copy(src_hbm.at[pl.ds(off, CH)], src_v)
        pltpu.sync_copy(dst_hbm.at[pl.ds(off, CH)], dst_v)
        pltpu.async_copy(p1tab_hbm.at[src_v], srows, sem).wait()
        pltpu.async_copy(p1tab_hbm.at[dst_v], drows, sem).wait()
        for g in range(5):
            ridx = iota16 + (g * 16)
            for hh in range(8):
                av = plsc.load_gather(srows, [ridx, jnp.full((16,), hh, _i32)])
                bv = plsc.load_gather(drows, [ridx, jnp.full((16,), 8 + hh, _i32)])
                ev = av + bv
                ev = jnp.where(ev >= 0, ev, 0.2 * ev)
                pv = jnp.exp(ev)
                pch[hh, pl.ds(g * 16, 16)] = pv
                plsc.store_scatter(pt, [ridx, jnp.full((16,), hh, _i32)], pv)
        pltpu.sync_copy(pch, p1_out.at[:, pl.ds(off, CH)])
        pltpu.sync_copy(pt, s_sh.at[dst_v], add=True)
        return c
    lax.fori_loop(0, NCHUNK, _chunk_a, 0)

    plsc.subcore_barrier()
    for k in range(5):
        pltpu.sync_copy(s_sh.at[pl.ds(rbase + k * 128, 128)],
                        s1_out.at[cid, pl.ds(rbase + k * 128, 128)])

    hts = [ht0, ht1, ht2, ht3]
    for pp in range(4):
        for k in range(5):
            pltpu.sync_copy(zb128, acc_sh.at[pl.ds(rbase + k * 128, 128)])
        plsc.subcore_barrier()

        def _chunk_b(ci, c, _pp=pp, _ht=hts[pp]):
            off = ebase + ci * CH
            pltpu.sync_copy(src_hbm.at[pl.ds(off, CH)], src_v)
            pltpu.sync_copy(dst_hbm.at[pl.ds(off, CH)], dst_v)
            pltpu.async_copy(_ht.at[src_v], rows, sem).wait()
            pltpu.sync_copy(p1_out.at[2 * _pp, pl.ds(off, CH)], pav)
            pltpu.sync_copy(p1_out.at[2 * _pp + 1, pl.ds(off, CH)], pbv)

            def _erow(e, cc):
                pa = plsc.load_gather(pav, [jnp.full((16,), e, _i32)])
                pb = plsc.load_gather(pbv, [jnp.full((16,), e, _i32)])
                for j in range(4):
                    rows[e, pl.ds(j * 16, 16)] = rows[e, pl.ds(j * 16, 16)] * pa
                for j in range(4, 8):
                    rows[e, pl.ds(j * 16, 16)] = rows[e, pl.ds(j * 16, 16)] * pb
                return cc
            lax.fori_loop(0, CH, _erow, 0)
            pltpu.sync_copy(rows, acc_sh.at[dst_v], add=True)
            return c
        lax.fori_loop(0, NCHUNK, _chunk_b, 0)

        plsc.subcore_barrier()
        for k in range(5):
            pltpu.sync_copy(acc_sh.at[pl.ds(rbase + k * 128, 128)],
                            acc1_out.at[cid, pp, pl.ds(rbase + k * 128, 128)])


def _sc1(src, dst, P1, ht0, ht1, ht2, ht3):
    mesh = plsc.VectorSubcoreMesh(core_axis_name="c", subcore_axis_name="s",
                                  num_cores=NC, num_subcores=NS)
    f = functools.partial(
        pl.kernel,
        out_type=[
            jax.ShapeDtypeStruct((8, E), _f32),
            jax.ShapeDtypeStruct((NC, NPAD, 16), _f32),
            jax.ShapeDtypeStruct((NC, 4, NPAD, 128), _f32),
        ],
        mesh=mesh,
        compiler_params=pltpu.CompilerParams(use_tc_tiling_on_sc=False, needs_layout_passes=False),
        scratch_types=[
            pltpu.VMEM_SHARED((NPAD, 16), _f32),
            pltpu.VMEM_SHARED((NPAD, 128), _f32),
            pltpu.VMEM((CH,), _i32),
            pltpu.VMEM((CH,), _i32),
            pltpu.VMEM((CH, 16), _f32),
            pltpu.VMEM((CH, 16), _f32),
            pltpu.VMEM((8, CH), _f32),
            pltpu.VMEM((CH, 16), _f32),
            pltpu.VMEM((CH, 128), _f32),
            pltpu.VMEM((CH,), _f32),
            pltpu.VMEM((CH,), _f32),
            pltpu.VMEM((128, 16), _f32),
            pltpu.VMEM((128, 128), _f32),
            pltpu.SemaphoreType.DMA,
        ],
    )(_sc1_body)
    return f(src, dst, P1, ht0, ht1, ht2, ht3)


def _stage3_body(acc1_ref, s1p_ref, ploop_ref, ht0, ht1, ht2, ht3,
                 b1_ref, w2_ref, as2_ref, ad2_ref,
                 h2t_ref, p2tab_ref):
    blk = ploop_ref.shape[0]
    s_tot = (s1p_ref[0, :, 0:8] + s1p_ref[1, :, 0:8] + ploop_ref[...])
    inv = 1.0 / (s_tot + 1e-16)
    hts = [ht0, ht1, ht2, ht3]
    cols = []
    for pp in range(4):
        acc = acc1_ref[0, pp] + acc1_ref[1, pp]
        hta = hts[pp][...]
        pw = jnp.concatenate(
            [jnp.broadcast_to(ploop_ref[:, 2 * pp:2 * pp + 1], (blk, 64)),
             jnp.broadcast_to(ploop_ref[:, 2 * pp + 1:2 * pp + 2], (blk, 64))],
            axis=1)
        iw = jnp.concatenate(
            [jnp.broadcast_to(inv[:, 2 * pp:2 * pp + 1], (blk, 64)),
             jnp.broadcast_to(inv[:, 2 * pp + 1:2 * pp + 2], (blk, 64))],
            axis=1)
        num = acc + pw * hta
        cols.append(num * iw)
    h1 = jnp.concatenate(cols, axis=1) + b1_ref[...]
    h1 = jnp.maximum(h1, 0.0)
    h2 = jnp.dot(h1, w2_ref[...], preferred_element_type=_f32)
    a_s2 = jnp.sum(h2 * as2_ref[...], axis=1, keepdims=True)
    a_d2 = jnp.sum(h2 * ad2_ref[...], axis=1, keepdims=True)
    el = a_s2 + a_d2
    el = jnp.where(el >= 0, el, 0.2 * el)
    pl2 = jnp.exp(el)
    h2t_ref[...] = jnp.concatenate(
        [h2, jnp.ones((blk, 1), _f32), jnp.zeros((blk, 15), _f32)], axis=1)
    p2tab_ref[...] = jnp.concatenate(
        [a_s2, a_d2, pl2, jnp.zeros((blk, 13), _f32)], axis=1)


def _stage3(acc1, s1p, ploop1, ht0, ht1, ht2, ht3, b1, W2, a_src2, a_dst2):
    grid = (N // _BLK,)
    return pl.pallas_call(
        _stage3_body,
        grid=grid,
        in_specs=[
            pl.BlockSpec((NC, 4, _BLK, 128), lambda i: (0, 0, i, 0)),
            pl.BlockSpec((NC, _BLK, 16), lambda i: (0, i, 0)),
            pl.BlockSpec((_BLK, 8), lambda i: (i, 0)),
            pl.BlockSpec((_BLK, 128), lambda i: (i, 0)),
            pl.BlockSpec((_BLK, 128), lambda i: (i, 0)),
            pl.BlockSpec((_BLK, 128), lambda i: (i, 0)),
            pl.BlockSpec((_BLK, 128), lambda i: (i, 0)),
            pl.BlockSpec((1, 512), lambda i: (0, 0)),
            pl.BlockSpec((512, 128), lambda i: (0, 0)),
            pl.BlockSpec((1, 128), lambda i: (0, 0)),
            pl.BlockSpec((1, 128), lambda i: (0, 0)),
        ],
        out_specs=[
            pl.BlockSpec((_BLK, 144), lambda i: (i, 0)),
            pl.BlockSpec((_BLK, 16), lambda i: (i, 0)),
        ],
        out_shape=[
            jax.ShapeDtypeStruct((N, 144), _f32),
            jax.ShapeDtypeStruct((N, 16), _f32),
        ],
    )(acc1, s1p, ploop1, ht0, ht1, ht2, ht3, b1, W2, a_src2, a_dst2)


def _sc2_body(src_hbm, dst_hbm, p2tab_hbm, h2t_hbm,
              acc2_out,
              acc_sh,
              src_v, dst_v, srows, drows, p2ch, rows,
              zb144, sem):
    cid = lax.axis_index("c")
    sid = lax.axis_index("s")
    wid = sid * NC + cid
    ebase = wid * EPW
    rbase = sid * RPT

    zv = jnp.zeros((16,), _f32)
    iota16 = lax.iota(_i32, 16)

    def _zb_init(i, c):
        for j in range(9):
            zb144[i, pl.ds(j * 16, 16)] = zv
        return c
    lax.fori_loop(0, 128, _zb_init, 0)

    for k in range(5):
        pltpu.sync_copy(zb144, acc_sh.at[pl.ds(rbase + k * 128, 128)])
    plsc.subcore_barrier()

    def _chunk(ci, c):
        off = ebase + ci * CH
        pltpu.sync_copy(src_hbm.at[pl.ds(off, CH)], src_v)
        pltpu.sync_copy(dst_hbm.at[pl.ds(off, CH)], dst_v)
        pltpu.async_copy(p2tab_hbm.at[src_v], srows, sem).wait()
        pltpu.async_copy(p2tab_hbm.at[dst_v], drows, sem).wait()
        pltpu.async_copy(h2t_hbm.at[src_v], rows, sem).wait()
        for g in range(5):
            ridx = iota16 + (g * 16)
            av = plsc.load_gather(srows, [ridx, jnp.full((16,), 0, _i32)])
            bv = plsc.load_gather(drows, [ridx, jnp.full((16,), 1, _i32)])
            ev = av + bv
            ev = jnp.where(ev >= 0, ev, 0.2 * ev)
            p2ch[pl.ds(g * 16, 16)] = jnp.exp(ev)

        def _erow(e, cc):
            pv = plsc.load_gather(p2ch, [jnp.full((16,), e, _i32)])
            for j in range(9):
                rows[e, pl.ds(j * 16, 16)] = rows[e, pl.ds(j * 16, 16)] * pv
            return cc
        lax.fori_loop(0, CH, _erow, 0)
        pltpu.sync_copy(rows, acc_sh.at[dst_v], add=True)
        return c
    lax.fori_loop(0, NCHUNK, _chunk, 0)

    plsc.subcore_barrier()
    for k in range(5):
        pltpu.sync_copy(acc_sh.at[pl.ds(rbase + k * 128, 128)],
                        acc2_out.at[cid, pl.ds(rbase + k * 128, 128)])


def _sc2(src, dst, P2, h2T):
    mesh = plsc.VectorSubcoreMesh(core_axis_name="c", subcore_axis_name="s",
                                  num_cores=NC, num_subcores=NS)
    f = functools.partial(
        pl.kernel,
        out_type=jax.ShapeDtypeStruct((NC, NPAD, 144), _f32),
        mesh=mesh,
        compiler_params=pltpu.CompilerParams(use_tc_tiling_on_sc=False, needs_layout_passes=False),
        scratch_types=[
            pltpu.VMEM_SHARED((NPAD, 144), _f32),
            pltpu.VMEM((CH,), _i32),
            pltpu.VMEM((CH,), _i32),
            pltpu.VMEM((CH, 16), _f32),
            pltpu.VMEM((CH, 16), _f32),
            pltpu.VMEM((CH,), _f32),
            pltpu.VMEM((CH, 144), _f32),
            pltpu.VMEM((128, 144), _f32),
            pltpu.SemaphoreType.DMA,
        ],
    )(_sc2_body)
    return f(src, dst, P2, h2T)


def _stage4_body(acc2_ref, h2t_ref, p2tab_ref, b2_ref, out_ref):
    accs = acc2_ref[0] + acc2_ref[1]
    pl2 = p2tab_ref[:, 2:3]
    num = accs[:, 0:128] + pl2 * h2t_ref[:, 0:128]
    s = accs[:, 128:129] + pl2
    o = num / (s + 1e-16) + b2_ref[...]
    m = jnp.max(o, axis=1, keepdims=True)
    z = o - m
    out_ref[...] = z - jnp.log(jnp.sum(jnp.exp(z), axis=1, keepdims=True))


def _stage4(acc2, h2T, P2, b2):
    grid = (N // _BLK,)
    return pl.pallas_call(
        _stage4_body,
        grid=grid,
        in_specs=[
            pl.BlockSpec((NC, _BLK, 144), lambda i: (0, i, 0)),
            pl.BlockSpec((_BLK, 144), lambda i: (i, 0)),
            pl.BlockSpec((_BLK, 16), lambda i: (i, 0)),
            pl.BlockSpec((1, 128), lambda i: (0, 0)),
        ],
        out_specs=pl.BlockSpec((_BLK, 128), lambda i: (i, 0)),
        out_shape=jax.ShapeDtypeStruct((N, D_OUT), _f32),
    )(acc2, h2T, P2, b2)


def kernel(x, edge_index, W1, a_src1, a_dst1, b1, W2, a_src2, a_dst2, b2):
    src = edge_index[0]
    dst = edge_index[1]
    eye8 = jnp.eye(HEADS, dtype=_f32)
    As1 = (a_src1[:, :, None] * eye8[:, None, :]).reshape(HEADS * HID, HEADS)
    Ad1 = (a_dst1[:, :, None] * eye8[:, None, :]).reshape(HEADS * HID, HEADS)

    ht0, ht1, ht2, ht3, P1, ploop1 = _stage1(x, W1, As1, Ad1)
    p1, s1p, acc1 = _sc1(src, dst, P1, ht0, ht1, ht2, ht3)
    h2T, P2 = _stage3(acc1, s1p, ploop1, ht0, ht1, ht2, ht3,
                      b1.reshape(1, -1), W2, a_src2, a_dst2)
    acc2 = _sc2(src, dst, P2, h2T)
    return _stage4(acc2, h2T, P2, b2.reshape(1, -1))


# speedup vs baseline: 45.6985x; 2.7225x over previous
import functools

import jax
import jax.numpy as jnp
from jax import lax
from jax.experimental import pallas as pl
from jax.experimental.pallas import tpu as pltpu
from jax.experimental.pallas import tpu_sc as plsc

N = 10000
E = 320000
D_IN = 128
HID = 64
HEADS = 8
D_OUT = 128

NC = 2
NS = 16
NW = NC * NS
EPW = E // NW
CH = 80
NCHUNK = EPW // CH
NPAD = 10240
RPT = NPAD // NS

_BLK = 2000

_f32 = jnp.float32
_i32 = jnp.int32


def _stage1_body(x_ref, w1_ref, as_ref, ad_ref,
                 ht0, ht1, ht2, ht3, p1tab, ploop):
    h = jnp.dot(x_ref[...], w1_ref[...], preferred_element_type=_f32)
    ht0[...] = h[:, 0:128]
    ht1[...] = h[:, 128:256]
    ht2[...] = h[:, 256:384]
    ht3[...] = h[:, 384:512]
    a_s = jnp.dot(h, as_ref[...], preferred_element_type=_f32)
    a_d = jnp.dot(h, ad_ref[...], preferred_element_type=_f32)
    p1tab[...] = jnp.concatenate([a_s, a_d], axis=1)
    el = a_s + a_d
    el = jnp.where(el >= 0, el, 0.2 * el)
    ploop[...] = jnp.exp(el)


def _stage1(x, W1, As1, Ad1):
    grid = (N // _BLK,)
    return pl.pallas_call(
        _stage1_body,
        grid=grid,
        in_specs=[
            pl.BlockSpec((_BLK, D_IN), lambda i: (i, 0)),
            pl.BlockSpec((D_IN, HEADS * HID), lambda i: (0, 0)),
            pl.BlockSpec((HEADS * HID, HEADS), lambda i: (0, 0)),
            pl.BlockSpec((HEADS * HID, HEADS), lambda i: (0, 0)),
        ],
        out_specs=[
            pl.BlockSpec((_BLK, 128), lambda i: (i, 0)),
            pl.BlockSpec((_BLK, 128), lambda i: (i, 0)),
            pl.BlockSpec((_BLK, 128), lambda i: (i, 0)),
            pl.BlockSpec((_BLK, 128), lambda i: (i, 0)),
            pl.BlockSpec((_BLK, 16), lambda i: (i, 0)),
            pl.BlockSpec((_BLK, 8), lambda i: (i, 0)),
        ],
        out_shape=[
            jax.ShapeDtypeStruct((N, 128), _f32),
            jax.ShapeDtypeStruct((N, 128), _f32),
            jax.ShapeDtypeStruct((N, 128), _f32),
            jax.ShapeDtypeStruct((N, 128), _f32),
            jax.ShapeDtypeStruct((N, 16), _f32),
            jax.ShapeDtypeStruct((N, 8), _f32),
        ],
    )(x, W1, As1, Ad1)


def _sc1_body(src_hbm, dst_hbm, p1tab_hbm, ht0, ht1, ht2, ht3,
              p1_out, s1_out, acc1_out,
              s_sh, acc_sh,
              src_v, dst_v, dsc_v, sa_rows, da_rows, pch, pt, rows,
              pav, pbv, zb16, zb128, sem_s, sem_d, sem_d2, sem_g):
    cid = lax.axis_index("c")
    sid = lax.axis_index("s")
    wid = sid * NC + cid
    ebase = wid * EPW
    rbase = sid * RPT

    zv = jnp.zeros((16,), _f32)
    iota16 = lax.iota(_i32, 16)

    def _zb_init(i, c):
        for j in range(8):
            zb128[i, pl.ds(j * 16, 16)] = zv
        return c
    lax.fori_loop(0, 16, _zb_init, 0)

    def _zb16_init(i, c):
        zb16[i, :] = zv
        return c
    lax.fori_loop(0, 64, _zb16_init, 0)

    def _pt_init(i, c):
        pt[i, :] = zv
        return c
    lax.fori_loop(0, CH, _pt_init, 0)

    for k in range(10):
        pltpu.sync_copy(zb16, s_sh.at[pl.ds(rbase + k * 64, 64)])
    plsc.subcore_barrier()

    def _issue_src(ci, b):
        pltpu.make_async_copy(src_hbm.at[pl.ds(ebase + ci * CH, CH)],
                              src_v[b], sem_s[b]).start()

    def _wait_src(ci, b):
        pltpu.make_async_copy(src_hbm.at[pl.ds(ebase + ci * CH, CH)],
                              src_v[b], sem_s[b]).wait()

    def _issue_dst(ci, b):
        pltpu.make_async_copy(dst_hbm.at[pl.ds(ebase + ci * CH, CH)],
                              dst_v[b], sem_d[b]).start()

    def _wait_dst(ci, b):
        pltpu.make_async_copy(dst_hbm.at[pl.ds(ebase + ci * CH, CH)],
                              dst_v[b], sem_d[b]).wait()

    def _issue_dsc(ci, b):
        pltpu.make_async_copy(dst_hbm.at[pl.ds(ebase + ci * CH, CH)],
                              dsc_v[b], sem_d2[b]).start()

    def _wait_dsc(ci, b):
        pltpu.make_async_copy(dst_hbm.at[pl.ds(ebase + ci * CH, CH)],
                              dsc_v[b], sem_d2[b]).wait()

    def _issue_ga(ci, b):
        pltpu.make_async_copy(p1tab_hbm.at[src_v[b]], sa_rows[b],
                              sem_g[b]).start()
        pltpu.make_async_copy(p1tab_hbm.at[dst_v[b]], da_rows[b],
                              sem_g[b]).start()

    def _wait_ga(ci, b):
        pltpu.make_async_copy(p1tab_hbm.at[src_v[b]], sa_rows[b],
                              sem_g[b]).wait()
        pltpu.make_async_copy(p1tab_hbm.at[dst_v[b]], da_rows[b],
                              sem_g[b]).wait()

    def _work_a(ci, b):
        off = ebase + ci * CH
        for g in range(5):
            ridx = iota16 + (g * 16)
            for hh in range(8):
                av = plsc.load_gather(sa_rows[b], [ridx, jnp.full((16,), hh, _i32)])
                bv = plsc.load_gather(da_rows[b], [ridx, jnp.full((16,), 8 + hh, _i32)])
                ev = av + bv
                ev = jnp.where(ev >= 0, ev, 0.2 * ev)
                pv = jnp.exp(ev)
                pch[hh, pl.ds(g * 16, 16)] = pv
                plsc.store_scatter(pt, [ridx, jnp.full((16,), hh, _i32)], pv)
        pltpu.sync_copy(pch, p1_out.at[:, pl.ds(off, CH)])
        _wait_dsc(ci, b)
        pltpu.sync_copy(pt, s_sh.at[dsc_v[b]], add=True)

    for b in (0, 1):
        _issue_src(b, b)
        _issue_dst(b, b)
        _issue_dsc(b, b)
    _wait_src(0, 0)
    _wait_dst(0, 0)
    _issue_ga(0, 0)

    def _pair_a(k, c):
        i = 2 * k + 1
        _wait_src(i, 1)
        _wait_dst(i, 1)
        _issue_ga(i, 1)
        _wait_ga(i - 1, 0)
        _issue_src(i + 1, 0)
        _issue_dst(i + 1, 0)
        _work_a(i - 1, 0)
        _issue_dsc(i + 1, 0)
        _wait_src(i + 1, 0)
        _wait_dst(i + 1, 0)
        _issue_ga(i + 1, 0)
        _wait_ga(i, 1)

        @pl.when(i + 2 < NCHUNK)
        def _():
            _issue_src(i + 2, 1)
            _issue_dst(i + 2, 1)
        _work_a(i, 1)

        @pl.when(i + 2 < NCHUNK)
        def _():
            _issue_dsc(i + 2, 1)
        return c
    lax.fori_loop(0, (NCHUNK - 1) // 2, _pair_a, 0)
    _wait_ga(NCHUNK - 1, 0)
    _work_a(NCHUNK - 1, 0)

    plsc.subcore_barrier()
    for k in range(5):
        pltpu.sync_copy(s_sh.at[pl.ds(rbase + k * 128, 128)],
                        s1_out.at[cid, pl.ds(rbase + k * 128, 128)])

    hts = [ht0, ht1, ht2, ht3]
    for pp in range(4):
        for k in range(40):
            pltpu.sync_copy(zb128, acc_sh.at[pl.ds(rbase + k * 16, 16)])
        plsc.subcore_barrier()
        ht = hts[pp]

        def _issue_gb(ci, b, _ht=ht, _pp=pp):
            pltpu.make_async_copy(_ht.at[src_v[b]], rows[b], sem_g[b]).start()
            pltpu.make_async_copy(
                p1_out.at[2 * _pp, pl.ds(ebase + ci * CH, CH)],
                pav[b], sem_g[b]).start()
            pltpu.make_async_copy(
                p1_out.at[2 * _pp + 1, pl.ds(ebase + ci * CH, CH)],
                pbv[b], sem_g[b]).start()

        def _wait_gb(ci, b, _ht=ht, _pp=pp):
            pltpu.make_async_copy(_ht.at[src_v[b]], rows[b], sem_g[b]).wait()
            pltpu.make_async_copy(
                p1_out.at[2 * _pp, pl.ds(ebase + ci * CH, CH)],
                pav[b], sem_g[b]).wait()
            pltpu.make_async_copy(
                p1_out.at[2 * _pp + 1, pl.ds(ebase + ci * CH, CH)],
                pbv[b], sem_g[b]).wait()

        def _work_b(ci, b):
            rb = rows[b]

            def _erow(e, cc):
                eg = jnp.full((16,), e, _i32)
                pa = plsc.load_gather(pav[b], [eg])
                pb = plsc.load_gather(pbv[b], [eg])
                for j in range(4):
                    rb[e, pl.ds(j * 16, 16)] = rb[e, pl.ds(j * 16, 16)] * pa
                for j in range(4, 8):
                    rb[e, pl.ds(j * 16, 16)] = rb[e, pl.ds(j * 16, 16)] * pb
                return cc
            lax.fori_loop(0, CH, _erow, 0)
            _wait_dsc(ci, b)
            pltpu.sync_copy(rb, acc_sh.at[dsc_v[b]], add=True)

        for b in (0, 1):
            _issue_src(b, b)
            _issue_dsc(b, b)
        _wait_src(0, 0)
        _issue_gb(0, 0)

        def _pair_b(k, c):
            i = 2 * k + 1
            _wait_src(i, 1)
            _issue_gb(i, 1)
            _wait_gb(i - 1, 0)
            _issue_src(i + 1, 0)
            _work_b(i - 1, 0)
            _issue_dsc(i + 1, 0)
            _wait_src(i + 1, 0)
            _issue_gb(i + 1, 0)
            _wait_gb(i, 1)

            @pl.when(i + 2 < NCHUNK)
            def _():
                _issue_src(i + 2, 1)
            _work_b(i, 1)

            @pl.when(i + 2 < NCHUNK)
            def _():
                _issue_dsc(i + 2, 1)
            return c
        lax.fori_loop(0, (NCHUNK - 1) // 2, _pair_b, 0)
        _wait_gb(NCHUNK - 1, 0)
        _work_b(NCHUNK - 1, 0)

        plsc.subcore_barrier()
        for k in range(5):
            pltpu.sync_copy(acc_sh.at[pl.ds(rbase + k * 128, 128)],
                            acc1_out.at[cid, pp, pl.ds(rbase + k * 128, 128)])


def _sc1(src, dst, P1, ht0, ht1, ht2, ht3):
    mesh = plsc.VectorSubcoreMesh(core_axis_name="c", subcore_axis_name="s",
                                  num_cores=NC, num_subcores=NS)
    f = functools.partial(
        pl.kernel,
        out_type=[
            jax.ShapeDtypeStruct((8, E), _f32),
            jax.ShapeDtypeStruct((NC, NPAD, 16), _f32),
            jax.ShapeDtypeStruct((NC, 4, NPAD, 128), _f32),
        ],
        mesh=mesh,
        compiler_params=pltpu.CompilerParams(use_tc_tiling_on_sc=False, needs_layout_passes=False),
        scratch_types=[
            pltpu.VMEM_SHARED((NPAD, 16), _f32),
            pltpu.VMEM_SHARED((NPAD, 128), _f32),
            [pltpu.VMEM((CH,), _i32), pltpu.VMEM((CH,), _i32)],
            [pltpu.VMEM((CH,), _i32), pltpu.VMEM((CH,), _i32)],
            [pltpu.VMEM((CH,), _i32), pltpu.VMEM((CH,), _i32)],
            [pltpu.VMEM((CH, 16), _f32), pltpu.VMEM((CH, 16), _f32)],
            [pltpu.VMEM((CH, 16), _f32), pltpu.VMEM((CH, 16), _f32)],
            pltpu.VMEM((8, CH), _f32),
            pltpu.VMEM((CH, 16), _f32),
            [pltpu.VMEM((CH, 128), _f32), pltpu.VMEM((CH, 128), _f32)],
            [pltpu.VMEM((CH,), _f32), pltpu.VMEM((CH,), _f32)],
            [pltpu.VMEM((CH,), _f32), pltpu.VMEM((CH,), _f32)],
            pltpu.VMEM((64, 16), _f32),
            pltpu.VMEM((16, 128), _f32),
            [pltpu.SemaphoreType.DMA, pltpu.SemaphoreType.DMA],
            [pltpu.SemaphoreType.DMA, pltpu.SemaphoreType.DMA],
            [pltpu.SemaphoreType.DMA, pltpu.SemaphoreType.DMA],
            [pltpu.SemaphoreType.DMA, pltpu.SemaphoreType.DMA],
        ],
    )(_sc1_body)
    return f(src, dst, P1, ht0, ht1, ht2, ht3)


def _stage3_body(acc1_ref, s1p_ref, ploop_ref, ht0, ht1, ht2, ht3,
                 b1_ref, w2_ref, as2_ref, ad2_ref,
                 h2t_ref, p2tab_ref):
    blk = ploop_ref.shape[0]
    s_tot = (s1p_ref[0, :, 0:8] + s1p_ref[1, :, 0:8] + ploop_ref[...])
    inv = 1.0 / (s_tot + 1e-16)
    hts = [ht0, ht1, ht2, ht3]
    cols = []
    for pp in range(4):
        acc = acc1_ref[0, pp] + acc1_ref[1, pp]
        hta = hts[pp][...]
        pw = jnp.concatenate(
            [jnp.broadcast_to(ploop_ref[:, 2 * pp:2 * pp + 1], (blk, 64)),
             jnp.broadcast_to(ploop_ref[:, 2 * pp + 1:2 * pp + 2], (blk, 64))],
            axis=1)
        iw = jnp.concatenate(
            [jnp.broadcast_to(inv[:, 2 * pp:2 * pp + 1], (blk, 64)),
             jnp.broadcast_to(inv[:, 2 * pp + 1:2 * pp + 2], (blk, 64))],
            axis=1)
        num = acc + pw * hta
        cols.append(num * iw)
    h1 = jnp.concatenate(cols, axis=1) + b1_ref[...]
    h1 = jnp.maximum(h1, 0.0)
    h2 = jnp.dot(h1, w2_ref[...], preferred_element_type=_f32)
    a_s2 = jnp.sum(h2 * as2_ref[...], axis=1, keepdims=True)
    a_d2 = jnp.sum(h2 * ad2_ref[...], axis=1, keepdims=True)
    el = a_s2 + a_d2
    el = jnp.where(el >= 0, el, 0.2 * el)
    pl2 = jnp.exp(el)
    h2t_ref[...] = jnp.concatenate(
        [h2, jnp.ones((blk, 1), _f32), jnp.zeros((blk, 15), _f32)], axis=1)
    p2tab_ref[...] = jnp.concatenate(
        [a_s2, a_d2, pl2, jnp.zeros((blk, 13), _f32)], axis=1)


def _stage3(acc1, s1p, ploop1, ht0, ht1, ht2, ht3, b1, W2, a_src2, a_dst2):
    grid = (N // _BLK,)
    return pl.pallas_call(
        _stage3_body,
        grid=grid,
        in_specs=[
            pl.BlockSpec((NC, 4, _BLK, 128), lambda i: (0, 0, i, 0)),
            pl.BlockSpec((NC, _BLK, 16), lambda i: (0, i, 0)),
            pl.BlockSpec((_BLK, 8), lambda i: (i, 0)),
            pl.BlockSpec((_BLK, 128), lambda i: (i, 0)),
            pl.BlockSpec((_BLK, 128), lambda i: (i, 0)),
            pl.BlockSpec((_BLK, 128), lambda i: (i, 0)),
            pl.BlockSpec((_BLK, 128), lambda i: (i, 0)),
            pl.BlockSpec((1, 512), lambda i: (0, 0)),
            pl.BlockSpec((512, 128), lambda i: (0, 0)),
            pl.BlockSpec((1, 128), lambda i: (0, 0)),
            pl.BlockSpec((1, 128), lambda i: (0, 0)),
        ],
        out_specs=[
            pl.BlockSpec((_BLK, 144), lambda i: (i, 0)),
            pl.BlockSpec((_BLK, 16), lambda i: (i, 0)),
        ],
        out_shape=[
            jax.ShapeDtypeStruct((N, 144), _f32),
            jax.ShapeDtypeStruct((N, 16), _f32),
        ],
    )(acc1, s1p, ploop1, ht0, ht1, ht2, ht3, b1, W2, a_src2, a_dst2)


def _sc2_body(src_hbm, dst_hbm, p2tab_hbm, h2t_hbm,
              acc2_out,
              acc_sh,
              src_v, dst_v, dsc_v, sa_rows, da_rows, p2ch, rows,
              zb144, sem_s, sem_d, sem_d2, sem_g):
    cid = lax.axis_index("c")
    sid = lax.axis_index("s")
    wid = sid * NC + cid
    ebase = wid * EPW
    rbase = sid * RPT

    zv = jnp.zeros((16,), _f32)
    iota16 = lax.iota(_i32, 16)

    def _zb_init(i, c):
        for j in range(9):
            zb144[i, pl.ds(j * 16, 16)] = zv
        return c
    lax.fori_loop(0, 16, _zb_init, 0)

    for k in range(40):
        pltpu.sync_copy(zb144, acc_sh.at[pl.ds(rbase + k * 16, 16)])
    plsc.subcore_barrier()

    def _issue_src(ci, b):
        pltpu.make_async_copy(src_hbm.at[pl.ds(ebase + ci * CH, CH)],
                              src_v[b], sem_s[b]).start()

    def _wait_src(ci, b):
        pltpu.make_async_copy(src_hbm.at[pl.ds(ebase + ci * CH, CH)],
                              src_v[b], sem_s[b]).wait()

    def _issue_dst(ci, b):
        pltpu.make_async_copy(dst_hbm.at[pl.ds(ebase + ci * CH, CH)],
                              dst_v[b], sem_d[b]).start()

    def _wait_dst(ci, b):
        pltpu.make_async_copy(dst_hbm.at[pl.ds(ebase + ci * CH, CH)],
                              dst_v[b], sem_d[b]).wait()

    def _issue_dsc(ci, b):
        pltpu.make_async_copy(dst_hbm.at[pl.ds(ebase + ci * CH, CH)],
                              dsc_v[b], sem_d2[b]).start()

    def _wait_dsc(ci, b):
        pltpu.make_async_copy(dst_hbm.at[pl.ds(ebase + ci * CH, CH)],
                              dsc_v[b], sem_d2[b]).wait()

    def _issue_g(ci, b):
        pltpu.make_async_copy(p2tab_hbm.at[src_v[b]], sa_rows[b],
                              sem_g[b]).start()
        pltpu.make_async_copy(p2tab_hbm.at[dst_v[b]], da_rows[b],
                              sem_g[b]).start()
        pltpu.make_async_copy(h2t_hbm.at[src_v[b]], rows[b],
                              sem_g[b]).start()

    def _wait_g(ci, b):
        pltpu.make_async_copy(p2tab_hbm.at[src_v[b]], sa_rows[b],
                              sem_g[b]).wait()
        pltpu.make_async_copy(p2tab_hbm.at[dst_v[b]], da_rows[b],
                              sem_g[b]).wait()
        pltpu.make_async_copy(h2t_hbm.at[src_v[b]], rows[b],
                              sem_g[b]).wait()

    def _work(ci, b):
        rb = rows[b]
        for g in range(5):
            ridx = iota16 + (g * 16)
            av = plsc.load_gather(sa_rows[b], [ridx, jnp.full((16,), 0, _i32)])
            bv = plsc.load_gather(da_rows[b], [ridx, jnp.full((16,), 1, _i32)])
            ev = av + bv
            ev = jnp.where(ev >= 0, ev, 0.2 * ev)
            p2ch[pl.ds(g * 16, 16)] = jnp.exp(ev)

        def _erow(e, cc):
            pv = plsc.load_gather(p2ch, [jnp.full((16,), e, _i32)])
            for j in range(9):
                rb[e, pl.ds(j * 16, 16)] = rb[e, pl.ds(j * 16, 16)] * pv
            return cc
        lax.fori_loop(0, CH, _erow, 0)
        _wait_dsc(ci, b)
        pltpu.sync_copy(rb, acc_sh.at[dsc_v[b]], add=True)

    for b in (0, 1):
        _issue_src(b, b)
        _issue_dst(b, b)
        _issue_dsc(b, b)
    _wait_src(0, 0)
    _wait_dst(0, 0)
    _issue_g(0, 0)

    def _pair(k, c):
        i = 2 * k + 1
        _wait_src(i, 1)
        _wait_dst(i, 1)
        _issue_g(i, 1)
        _wait_g(i - 1, 0)
        _issue_src(i + 1, 0)
        _issue_dst(i + 1, 0)
        _work(i - 1, 0)
        _issue_dsc(i + 1, 0)
        _wait_src(i + 1, 0)
        _wait_dst(i + 1, 0)
        _issue_g(i + 1, 0)
        _wait_g(i, 1)

        @pl.when(i + 2 < NCHUNK)
        def _():
            _issue_src(i + 2, 1)
            _issue_dst(i + 2, 1)
        _work(i, 1)

        @pl.when(i + 2 < NCHUNK)
        def _():
            _issue_dsc(i + 2, 1)
        return c
    lax.fori_loop(0, (NCHUNK - 1) // 2, _pair, 0)
    _wait_g(NCHUNK - 1, 0)
    _work(NCHUNK - 1, 0)

    plsc.subcore_barrier()
    for k in range(5):
        pltpu.sync_copy(acc_sh.at[pl.ds(rbase + k * 128, 128)],
                        acc2_out.at[cid, pl.ds(rbase + k * 128, 128)])


def _sc2(src, dst, P2, h2T):
    mesh = plsc.VectorSubcoreMesh(core_axis_name="c", subcore_axis_name="s",
                                  num_cores=NC, num_subcores=NS)
    f = functools.partial(
        pl.kernel,
        out_type=jax.ShapeDtypeStruct((NC, NPAD, 144), _f32),
        mesh=mesh,
        compiler_params=pltpu.CompilerParams(use_tc_tiling_on_sc=False, needs_layout_passes=False),
        scratch_types=[
            pltpu.VMEM_SHARED((NPAD, 144), _f32),
            [pltpu.VMEM((CH,), _i32), pltpu.VMEM((CH,), _i32)],
            [pltpu.VMEM((CH,), _i32), pltpu.VMEM((CH,), _i32)],
            [pltpu.VMEM((CH,), _i32), pltpu.VMEM((CH,), _i32)],
            [pltpu.VMEM((CH, 16), _f32), pltpu.VMEM((CH, 16), _f32)],
            [pltpu.VMEM((CH, 16), _f32), pltpu.VMEM((CH, 16), _f32)],
            pltpu.VMEM((CH,), _f32),
            [pltpu.VMEM((CH, 144), _f32), pltpu.VMEM((CH, 144), _f32)],
            pltpu.VMEM((16, 144), _f32),
            [pltpu.SemaphoreType.DMA, pltpu.SemaphoreType.DMA],
            [pltpu.SemaphoreType.DMA, pltpu.SemaphoreType.DMA],
            [pltpu.SemaphoreType.DMA, pltpu.SemaphoreType.DMA],
            [pltpu.SemaphoreType.DMA, pltpu.SemaphoreType.DMA],
        ],
    )(_sc2_body)
    return f(src, dst, P2, h2T)


def _stage4_body(acc2_ref, h2t_ref, p2tab_ref, b2_ref, out_ref):
    accs = acc2_ref[0] + acc2_ref[1]
    pl2 = p2tab_ref[:, 2:3]
    num = accs[:, 0:128] + pl2 * h2t_ref[:, 0:128]
    s = accs[:, 128:129] + pl2
    o = num / (s + 1e-16) + b2_ref[...]
    m = jnp.max(o, axis=1, keepdims=True)
    z = o - m
    out_ref[...] = z - jnp.log(jnp.sum(jnp.exp(z), axis=1, keepdims=True))


def _stage4(acc2, h2T, P2, b2):
    grid = (N // _BLK,)
    return pl.pallas_call(
        _stage4_body,
        grid=grid,
        in_specs=[
            pl.BlockSpec((NC, _BLK, 144), lambda i: (0, i, 0)),
            pl.BlockSpec((_BLK, 144), lambda i: (i, 0)),
            pl.BlockSpec((_BLK, 16), lambda i: (i, 0)),
            pl.BlockSpec((1, 128), lambda i: (0, 0)),
        ],
        out_specs=pl.BlockSpec((_BLK, 128), lambda i: (i, 0)),
        out_shape=jax.ShapeDtypeStruct((N, D_OUT), _f32),
    )(acc2, h2T, P2, b2)


def kernel(x, edge_index, W1, a_src1, a_dst1, b1, W2, a_src2, a_dst2, b2):
    src = edge_index[0]
    dst = edge_index[1]
    eye8 = jnp.eye(HEADS, dtype=_f32)
    As1 = (a_src1[:, :, None] * eye8[:, None, :]).reshape(HEADS * HID, HEADS)
    Ad1 = (a_dst1[:, :, None] * eye8[:, None, :]).reshape(HEADS * HID, HEADS)

    ht0, ht1, ht2, ht3, P1, ploop1 = _stage1(x, W1, As1, Ad1)
    p1, s1p, acc1 = _sc1(src, dst, P1, ht0, ht1, ht2, ht3)
    h2T, P2 = _stage3(acc1, s1p, ploop1, ht0, ht1, ht2, ht3,
                      b1.reshape(1, -1), W2, a_src2, a_dst2)
    acc2 = _sc2(src, dst, P2, h2T)
    return _stage4(acc2, h2T, P2, b2.reshape(1, -1))


# speedup vs baseline: 52.1573x; 1.1413x over previous
import functools

import jax
import jax.numpy as jnp
from jax import lax
from jax.experimental import pallas as pl
from jax.experimental.pallas import tpu as pltpu
from jax.experimental.pallas import tpu_sc as plsc

N = 10000
E = 320000
D_IN = 128
HID = 64
HEADS = 8
D_OUT = 128

NC = 2
NS = 16
NW = NC * NS
EPW = E // NW
CH = 80
NCHUNK = EPW // CH
NPAD = 10240
RPT = NPAD // NS

_BLK = 2000

_f32 = jnp.float32
_i32 = jnp.int32


def _stage1_body(x_ref, w1_ref, as_ref, ad_ref,
                 ht0, ht1, ht2, ht3, p1tab, ploop):
    h = jnp.dot(x_ref[...], w1_ref[...], preferred_element_type=_f32)
    ht0[...] = h[:, 0:128]
    ht1[...] = h[:, 128:256]
    ht2[...] = h[:, 256:384]
    ht3[...] = h[:, 384:512]
    a_s = jnp.dot(h, as_ref[...], preferred_element_type=_f32)
    a_d = jnp.dot(h, ad_ref[...], preferred_element_type=_f32)
    p1tab[...] = jnp.concatenate([a_s, a_d], axis=1)
    el = a_s + a_d
    el = jnp.where(el >= 0, el, 0.2 * el)
    ploop[...] = jnp.exp(el)


def _stage1(x, W1, As1, Ad1):
    grid = (N // _BLK,)
    return pl.pallas_call(
        _stage1_body,
        grid=grid,
        in_specs=[
            pl.BlockSpec((_BLK, D_IN), lambda i: (i, 0)),
            pl.BlockSpec((D_IN, HEADS * HID), lambda i: (0, 0)),
            pl.BlockSpec((HEADS * HID, HEADS), lambda i: (0, 0)),
            pl.BlockSpec((HEADS * HID, HEADS), lambda i: (0, 0)),
        ],
        out_specs=[
            pl.BlockSpec((_BLK, 128), lambda i: (i, 0)),
            pl.BlockSpec((_BLK, 128), lambda i: (i, 0)),
            pl.BlockSpec((_BLK, 128), lambda i: (i, 0)),
            pl.BlockSpec((_BLK, 128), lambda i: (i, 0)),
            pl.BlockSpec((_BLK, 16), lambda i: (i, 0)),
            pl.BlockSpec((_BLK, 8), lambda i: (i, 0)),
        ],
        out_shape=[
            jax.ShapeDtypeStruct((N, 128), _f32),
            jax.ShapeDtypeStruct((N, 128), _f32),
            jax.ShapeDtypeStruct((N, 128), _f32),
            jax.ShapeDtypeStruct((N, 128), _f32),
            jax.ShapeDtypeStruct((N, 16), _f32),
            jax.ShapeDtypeStruct((N, 8), _f32),
        ],
    )(x, W1, As1, Ad1)


def _sc1_body(src_hbm, dst_hbm, p1tab_hbm, ht0, ht1, ht2, ht3,
              p1_out, s1_out, acc1_out,
              s_sh, acc_sh,
              src_v, dst_v, dsc_v, sa_rows, da_rows, pch, pt, rows,
              pav, pbv, zb16, zb128, sem_s, sem_d, sem_d2, sem_g):
    cid = lax.axis_index("c")
    sid = lax.axis_index("s")
    wid = sid * NC + cid
    ebase = wid * EPW
    rbase = sid * RPT

    zv = jnp.zeros((16,), _f32)
    iota16 = lax.iota(_i32, 16)

    def _zb_init(i, c):
        for j in range(8):
            zb128[i, pl.ds(j * 16, 16)] = zv
        return c
    lax.fori_loop(0, 16, _zb_init, 0)

    def _zb16_init(i, c):
        zb16[i, :] = zv
        return c
    lax.fori_loop(0, 64, _zb16_init, 0)

    def _pt_init(i, c):
        pt[i, :] = zv
        return c
    lax.fori_loop(0, CH, _pt_init, 0)

    for k in range(10):
        pltpu.sync_copy(zb16, s_sh.at[pl.ds(rbase + k * 64, 64)])
    plsc.subcore_barrier()

    def _issue_src(ci, b):
        pltpu.make_async_copy(src_hbm.at[pl.ds(ebase + ci * CH, CH)],
                              src_v[b], sem_s[b]).start()

    def _wait_src(ci, b):
        pltpu.make_async_copy(src_hbm.at[pl.ds(ebase + ci * CH, CH)],
                              src_v[b], sem_s[b]).wait()

    def _issue_dst(ci, b):
        pltpu.make_async_copy(dst_hbm.at[pl.ds(ebase + ci * CH, CH)],
                              dst_v[b], sem_d[b]).start()

    def _wait_dst(ci, b):
        pltpu.make_async_copy(dst_hbm.at[pl.ds(ebase + ci * CH, CH)],
                              dst_v[b], sem_d[b]).wait()

    def _issue_dsc(ci, b):
        pltpu.make_async_copy(dst_hbm.at[pl.ds(ebase + ci * CH, CH)],
                              dsc_v[b], sem_d2[b]).start()

    def _wait_dsc(ci, b):
        pltpu.make_async_copy(dst_hbm.at[pl.ds(ebase + ci * CH, CH)],
                              dsc_v[b], sem_d2[b]).wait()

    def _issue_ga(ci, b):
        pltpu.make_async_copy(p1tab_hbm.at[src_v[b]], sa_rows[b],
                              sem_g[b]).start()
        pltpu.make_async_copy(p1tab_hbm.at[dst_v[b]], da_rows[b],
                              sem_g[b]).start()

    def _wait_ga(ci, b):
        pltpu.make_async_copy(p1tab_hbm.at[src_v[b]], sa_rows[b],
                              sem_g[b]).wait()
        pltpu.make_async_copy(p1tab_hbm.at[dst_v[b]], da_rows[b],
                              sem_g[b]).wait()

    def _work_a(ci, b):
        off = ebase + ci * CH
        for g in range(5):
            ridx = iota16 + (g * 16)
            for hh in range(8):
                av = plsc.load_gather(sa_rows[b], [ridx, jnp.full((16,), hh, _i32)])
                bv = plsc.load_gather(da_rows[b], [ridx, jnp.full((16,), 8 + hh, _i32)])
                ev = av + bv
                ev = jnp.where(ev >= 0, ev, 0.2 * ev)
                pv = jnp.exp(ev)
                pch[hh, pl.ds(g * 16, 16)] = pv
                plsc.store_scatter(pt, [ridx, jnp.full((16,), hh, _i32)], pv)
        pltpu.sync_copy(pch, p1_out.at[:, pl.ds(off, CH)])
        _wait_dsc(ci, b)
        pltpu.sync_copy(pt, s_sh.at[dsc_v[b]], add=True)

    for b in (0, 1):
        _issue_src(b, b)
        _issue_dst(b, b)
        _issue_dsc(b, b)
    _wait_src(0, 0)
    _wait_dst(0, 0)
    _issue_ga(0, 0)

    def _pair_a(k, c):
        i = 2 * k + 1
        _wait_src(i, 1)
        _wait_dst(i, 1)
        _issue_ga(i, 1)
        _wait_ga(i - 1, 0)
        _issue_src(i + 1, 0)
        _issue_dst(i + 1, 0)
        _work_a(i - 1, 0)
        _issue_dsc(i + 1, 0)
        _wait_src(i + 1, 0)
        _wait_dst(i + 1, 0)
        _issue_ga(i + 1, 0)
        _wait_ga(i, 1)

        @pl.when(i + 2 < NCHUNK)
        def _():
            _issue_src(i + 2, 1)
            _issue_dst(i + 2, 1)
        _work_a(i, 1)

        @pl.when(i + 2 < NCHUNK)
        def _():
            _issue_dsc(i + 2, 1)
        return c
    lax.fori_loop(0, (NCHUNK - 1) // 2, _pair_a, 0)
    _wait_ga(NCHUNK - 1, 0)
    _work_a(NCHUNK - 1, 0)

    plsc.subcore_barrier()
    for k in range(5):
        pltpu.sync_copy(s_sh.at[pl.ds(rbase + k * 128, 128)],
                        s1_out.at[cid, pl.ds(rbase + k * 128, 128)])

    hts = [ht0, ht1, ht2, ht3]
    for pp in range(4):
        for k in range(40):
            pltpu.sync_copy(zb128, acc_sh.at[pl.ds(rbase + k * 16, 16)])
        plsc.subcore_barrier()
        ht = hts[pp]

        def _issue_gb(ci, b, _ht=ht, _pp=pp):
            pltpu.make_async_copy(_ht.at[src_v[b]], rows[b], sem_g[b]).start()
            pltpu.make_async_copy(
                p1_out.at[2 * _pp, pl.ds(ebase + ci * CH, CH)],
                pav[b], sem_g[b]).start()
            pltpu.make_async_copy(
                p1_out.at[2 * _pp + 1, pl.ds(ebase + ci * CH, CH)],
                pbv[b], sem_g[b]).start()

        def _wait_gb(ci, b, _ht=ht, _pp=pp):
            pltpu.make_async_copy(_ht.at[src_v[b]], rows[b], sem_g[b]).wait()
            pltpu.make_async_copy(
                p1_out.at[2 * _pp, pl.ds(ebase + ci * CH, CH)],
                pav[b], sem_g[b]).wait()
            pltpu.make_async_copy(
                p1_out.at[2 * _pp + 1, pl.ds(ebase + ci * CH, CH)],
                pbv[b], sem_g[b]).wait()

        def _work_b(ci, b):
            rb = rows[b]

            @plsc.parallel_loop(0, CH, 1, unroll=8)
            def _erow(e):
                eg = jnp.full((16,), e, _i32)
                pa = plsc.load_gather(pav[b], [eg])
                pb = plsc.load_gather(pbv[b], [eg])
                for j in range(4):
                    rb[e, pl.ds(j * 16, 16)] = rb[e, pl.ds(j * 16, 16)] * pa
                for j in range(4, 8):
                    rb[e, pl.ds(j * 16, 16)] = rb[e, pl.ds(j * 16, 16)] * pb
            _wait_dsc(ci, b)
            pltpu.sync_copy(rb, acc_sh.at[dsc_v[b]], add=True)

        for b in (0, 1):
            _issue_src(b, b)
            _issue_dsc(b, b)
        _wait_src(0, 0)
        _issue_gb(0, 0)

        def _pair_b(k, c):
            i = 2 * k + 1
            _wait_src(i, 1)
            _issue_gb(i, 1)
            _wait_gb(i - 1, 0)
            _issue_src(i + 1, 0)
            _work_b(i - 1, 0)
            _issue_dsc(i + 1, 0)
            _wait_src(i + 1, 0)
            _issue_gb(i + 1, 0)
            _wait_gb(i, 1)

            @pl.when(i + 2 < NCHUNK)
            def _():
                _issue_src(i + 2, 1)
            _work_b(i, 1)

            @pl.when(i + 2 < NCHUNK)
            def _():
                _issue_dsc(i + 2, 1)
            return c
        lax.fori_loop(0, (NCHUNK - 1) // 2, _pair_b, 0)
        _wait_gb(NCHUNK - 1, 0)
        _work_b(NCHUNK - 1, 0)

        plsc.subcore_barrier()
        for k in range(5):
            pltpu.sync_copy(acc_sh.at[pl.ds(rbase + k * 128, 128)],
                            acc1_out.at[cid, pp, pl.ds(rbase + k * 128, 128)])


def _sc1(src, dst, P1, ht0, ht1, ht2, ht3):
    mesh = plsc.VectorSubcoreMesh(core_axis_name="c", subcore_axis_name="s",
                                  num_cores=NC, num_subcores=NS)
    f = functools.partial(
        pl.kernel,
        out_type=[
            jax.ShapeDtypeStruct((8, E), _f32),
            jax.ShapeDtypeStruct((NC, NPAD, 16), _f32),
            jax.ShapeDtypeStruct((NC, 4, NPAD, 128), _f32),
        ],
        mesh=mesh,
        compiler_params=pltpu.CompilerParams(use_tc_tiling_on_sc=False, needs_layout_passes=False),
        scratch_types=[
            pltpu.VMEM_SHARED((NPAD, 16), _f32),
            pltpu.VMEM_SHARED((NPAD, 128), _f32),
            [pltpu.VMEM((CH,), _i32), pltpu.VMEM((CH,), _i32)],
            [pltpu.VMEM((CH,), _i32), pltpu.VMEM((CH,), _i32)],
            [pltpu.VMEM((CH,), _i32), pltpu.VMEM((CH,), _i32)],
            [pltpu.VMEM((CH, 16), _f32), pltpu.VMEM((CH, 16), _f32)],
            [pltpu.VMEM((CH, 16), _f32), pltpu.VMEM((CH, 16), _f32)],
            pltpu.VMEM((8, CH), _f32),
            pltpu.VMEM((CH, 16), _f32),
            [pltpu.VMEM((CH, 128), _f32), pltpu.VMEM((CH, 128), _f32)],
            [pltpu.VMEM((CH,), _f32), pltpu.VMEM((CH,), _f32)],
            [pltpu.VMEM((CH,), _f32), pltpu.VMEM((CH,), _f32)],
            pltpu.VMEM((64, 16), _f32),
            pltpu.VMEM((16, 128), _f32),
            [pltpu.SemaphoreType.DMA, pltpu.SemaphoreType.DMA],
            [pltpu.SemaphoreType.DMA, pltpu.SemaphoreType.DMA],
            [pltpu.SemaphoreType.DMA, pltpu.SemaphoreType.DMA],
            [pltpu.SemaphoreType.DMA, pltpu.SemaphoreType.DMA],
        ],
    )(_sc1_body)
    return f(src, dst, P1, ht0, ht1, ht2, ht3)


def _stage3_body(acc1_ref, s1p_ref, ploop_ref, ht0, ht1, ht2, ht3,
                 b1_ref, w2_ref, as2_ref, ad2_ref,
                 h2t_ref, p2tab_ref):
    blk = ploop_ref.shape[0]
    s_tot = (s1p_ref[0, :, 0:8] + s1p_ref[1, :, 0:8] + ploop_ref[...])
    inv = 1.0 / (s_tot + 1e-16)
    hts = [ht0, ht1, ht2, ht3]
    cols = []
    for pp in range(4):
        acc = acc1_ref[0, pp] + acc1_ref[1, pp]
        hta = hts[pp][...]
        pw = jnp.concatenate(
            [jnp.broadcast_to(ploop_ref[:, 2 * pp:2 * pp + 1], (blk, 64)),
             jnp.broadcast_to(ploop_ref[:, 2 * pp + 1:2 * pp + 2], (blk, 64))],
            axis=1)
        iw = jnp.concatenate(
            [jnp.broadcast_to(inv[:, 2 * pp:2 * pp + 1], (blk, 64)),
             jnp.broadcast_to(inv[:, 2 * pp + 1:2 * pp + 2], (blk, 64))],
            axis=1)
        num = acc + pw * hta
        cols.append(num * iw)
    h1 = jnp.concatenate(cols, axis=1) + b1_ref[...]
    h1 = jnp.maximum(h1, 0.0)
    h2 = jnp.dot(h1, w2_ref[...], preferred_element_type=_f32)
    a_s2 = jnp.sum(h2 * as2_ref[...], axis=1, keepdims=True)
    a_d2 = jnp.sum(h2 * ad2_ref[...], axis=1, keepdims=True)
    el = a_s2 + a_d2
    el = jnp.where(el >= 0, el, 0.2 * el)
    pl2 = jnp.exp(el)
    h2t_ref[...] = jnp.concatenate(
        [h2, jnp.ones((blk, 1), _f32), jnp.zeros((blk, 15), _f32)], axis=1)
    p2tab_ref[...] = jnp.concatenate(
        [a_s2, a_d2, pl2, jnp.zeros((blk, 13), _f32)], axis=1)


def _stage3(acc1, s1p, ploop1, ht0, ht1, ht2, ht3, b1, W2, a_src2, a_dst2):
    grid = (N // _BLK,)
    return pl.pallas_call(
        _stage3_body,
        grid=grid,
        in_specs=[
            pl.BlockSpec((NC, 4, _BLK, 128), lambda i: (0, 0, i, 0)),
            pl.BlockSpec((NC, _BLK, 16), lambda i: (0, i, 0)),
            pl.BlockSpec((_BLK, 8), lambda i: (i, 0)),
            pl.BlockSpec((_BLK, 128), lambda i: (i, 0)),
            pl.BlockSpec((_BLK, 128), lambda i: (i, 0)),
            pl.BlockSpec((_BLK, 128), lambda i: (i, 0)),
            pl.BlockSpec((_BLK, 128), lambda i: (i, 0)),
            pl.BlockSpec((1, 512), lambda i: (0, 0)),
            pl.BlockSpec((512, 128), lambda i: (0, 0)),
            pl.BlockSpec((1, 128), lambda i: (0, 0)),
            pl.BlockSpec((1, 128), lambda i: (0, 0)),
        ],
        out_specs=[
            pl.BlockSpec((_BLK, 144), lambda i: (i, 0)),
            pl.BlockSpec((_BLK, 16), lambda i: (i, 0)),
        ],
        out_shape=[
            jax.ShapeDtypeStruct((N, 144), _f32),
            jax.ShapeDtypeStruct((N, 16), _f32),
        ],
    )(acc1, s1p, ploop1, ht0, ht1, ht2, ht3, b1, W2, a_src2, a_dst2)


def _sc2_body(src_hbm, dst_hbm, p2tab_hbm, h2t_hbm,
              acc2_out,
              acc_sh,
              src_v, dst_v, dsc_v, sa_rows, da_rows, p2ch, rows,
              zb144, sem_s, sem_d, sem_d2, sem_g):
    cid = lax.axis_index("c")
    sid = lax.axis_index("s")
    wid = sid * NC + cid
    ebase = wid * EPW
    rbase = sid * RPT

    zv = jnp.zeros((16,), _f32)
    iota16 = lax.iota(_i32, 16)

    def _zb_init(i, c):
        for j in range(9):
            zb144[i, pl.ds(j * 16, 16)] = zv
        return c
    lax.fori_loop(0, 16, _zb_init, 0)

    for k in range(40):
        pltpu.sync_copy(zb144, acc_sh.at[pl.ds(rbase + k * 16, 16)])
    plsc.subcore_barrier()

    def _issue_src(ci, b):
        pltpu.make_async_copy(src_hbm.at[pl.ds(ebase + ci * CH, CH)],
                              src_v[b], sem_s[b]).start()

    def _wait_src(ci, b):
        pltpu.make_async_copy(src_hbm.at[pl.ds(ebase + ci * CH, CH)],
                              src_v[b], sem_s[b]).wait()

    def _issue_dst(ci, b):
        pltpu.make_async_copy(dst_hbm.at[pl.ds(ebase + ci * CH, CH)],
                              dst_v[b], sem_d[b]).start()

    def _wait_dst(ci, b):
        pltpu.make_async_copy(dst_hbm.at[pl.ds(ebase + ci * CH, CH)],
                              dst_v[b], sem_d[b]).wait()

    def _issue_dsc(ci, b):
        pltpu.make_async_copy(dst_hbm.at[pl.ds(ebase + ci * CH, CH)],
                              dsc_v[b], sem_d2[b]).start()

    def _wait_dsc(ci, b):
        pltpu.make_async_copy(dst_hbm.at[pl.ds(ebase + ci * CH, CH)],
                              dsc_v[b], sem_d2[b]).wait()

    def _issue_g(ci, b):
        pltpu.make_async_copy(p2tab_hbm.at[src_v[b]], sa_rows[b],
                              sem_g[b]).start()
        pltpu.make_async_copy(p2tab_hbm.at[dst_v[b]], da_rows[b],
                              sem_g[b]).start()
        pltpu.make_async_copy(h2t_hbm.at[src_v[b]], rows[b],
                              sem_g[b]).start()

    def _wait_g(ci, b):
        pltpu.make_async_copy(p2tab_hbm.at[src_v[b]], sa_rows[b],
                              sem_g[b]).wait()
        pltpu.make_async_copy(p2tab_hbm.at[dst_v[b]], da_rows[b],
                              sem_g[b]).wait()
        pltpu.make_async_copy(h2t_hbm.at[src_v[b]], rows[b],
                              sem_g[b]).wait()

    def _work(ci, b):
        rb = rows[b]
        for g in range(5):
            ridx = iota16 + (g * 16)
            av = plsc.load_gather(sa_rows[b], [ridx, jnp.full((16,), 0, _i32)])
            bv = plsc.load_gather(da_rows[b], [ridx, jnp.full((16,), 1, _i32)])
            ev = av + bv
            ev = jnp.where(ev >= 0, ev, 0.2 * ev)
            p2ch[pl.ds(g * 16, 16)] = jnp.exp(ev)

        @plsc.parallel_loop(0, CH, 1, unroll=8)
        def _erow(e):
            pv = plsc.load_gather(p2ch, [jnp.full((16,), e, _i32)])
            for j in range(9):
                rb[e, pl.ds(j * 16, 16)] = rb[e, pl.ds(j * 16, 16)] * pv
        _wait_dsc(ci, b)
        pltpu.sync_copy(rb, acc_sh.at[dsc_v[b]], add=True)

    for b in (0, 1):
        _issue_src(b, b)
        _issue_dst(b, b)
        _issue_dsc(b, b)
    _wait_src(0, 0)
    _wait_dst(0, 0)
    _issue_g(0, 0)

    def _pair(k, c):
        i = 2 * k + 1
        _wait_src(i, 1)
        _wait_dst(i, 1)
        _issue_g(i, 1)
        _wait_g(i - 1, 0)
        _issue_src(i + 1, 0)
        _issue_dst(i + 1, 0)
        _work(i - 1, 0)
        _issue_dsc(i + 1, 0)
        _wait_src(i + 1, 0)
        _wait_dst(i + 1, 0)
        _issue_g(i + 1, 0)
        _wait_g(i, 1)

        @pl.when(i + 2 < NCHUNK)
        def _():
            _issue_src(i + 2, 1)
            _issue_dst(i + 2, 1)
        _work(i, 1)

        @pl.when(i + 2 < NCHUNK)
        def _():
            _issue_dsc(i + 2, 1)
        return c
    lax.fori_loop(0, (NCHUNK - 1) // 2, _pair, 0)
    _wait_g(NCHUNK - 1, 0)
    _work(NCHUNK - 1, 0)

    plsc.subcore_barrier()
    for k in range(5):
        pltpu.sync_copy(acc_sh.at[pl.ds(rbase + k * 128, 128)],
                        acc2_out.at[cid, pl.ds(rbase + k * 128, 128)])


def _sc2(src, dst, P2, h2T):
    mesh = plsc.VectorSubcoreMesh(core_axis_name="c", subcore_axis_name="s",
                                  num_cores=NC, num_subcores=NS)
    f = functools.partial(
        pl.kernel,
        out_type=jax.ShapeDtypeStruct((NC, NPAD, 144), _f32),
        mesh=mesh,
        compiler_params=pltpu.CompilerParams(use_tc_tiling_on_sc=False, needs_layout_passes=False),
        scratch_types=[
            pltpu.VMEM_SHARED((NPAD, 144), _f32),
            [pltpu.VMEM((CH,), _i32), pltpu.VMEM((CH,), _i32)],
            [pltpu.VMEM((CH,), _i32), pltpu.VMEM((CH,), _i32)],
            [pltpu.VMEM((CH,), _i32), pltpu.VMEM((CH,), _i32)],
            [pltpu.VMEM((CH, 16), _f32), pltpu.VMEM((CH, 16), _f32)],
            [pltpu.VMEM((CH, 16), _f32), pltpu.VMEM((CH, 16), _f32)],
            pltpu.VMEM((CH,), _f32),
            [pltpu.VMEM((CH, 144), _f32), pltpu.VMEM((CH, 144), _f32)],
            pltpu.VMEM((16, 144), _f32),
            [pltpu.SemaphoreType.DMA, pltpu.SemaphoreType.DMA],
            [pltpu.SemaphoreType.DMA, pltpu.SemaphoreType.DMA],
            [pltpu.SemaphoreType.DMA, pltpu.SemaphoreType.DMA],
            [pltpu.SemaphoreType.DMA, pltpu.SemaphoreType.DMA],
        ],
    )(_sc2_body)
    return f(src, dst, P2, h2T)


def _stage4_body(acc2_ref, h2t_ref, p2tab_ref, b2_ref, out_ref):
    accs = acc2_ref[0] + acc2_ref[1]
    pl2 = p2tab_ref[:, 2:3]
    num = accs[:, 0:128] + pl2 * h2t_ref[:, 0:128]
    s = accs[:, 128:129] + pl2
    o = num / (s + 1e-16) + b2_ref[...]
    m = jnp.max(o, axis=1, keepdims=True)
    z = o - m
    out_ref[...] = z - jnp.log(jnp.sum(jnp.exp(z), axis=1, keepdims=True))


def _stage4(acc2, h2T, P2, b2):
    grid = (N // _BLK,)
    return pl.pallas_call(
        _stage4_body,
        grid=grid,
        in_specs=[
            pl.BlockSpec((NC, _BLK, 144), lambda i: (0, i, 0)),
            pl.BlockSpec((_BLK, 144), lambda i: (i, 0)),
            pl.BlockSpec((_BLK, 16), lambda i: (i, 0)),
            pl.BlockSpec((1, 128), lambda i: (0, 0)),
        ],
        out_specs=pl.BlockSpec((_BLK, 128), lambda i: (i, 0)),
        out_shape=jax.ShapeDtypeStruct((N, D_OUT), _f32),
    )(acc2, h2T, P2, b2)


def kernel(x, edge_index, W1, a_src1, a_dst1, b1, W2, a_src2, a_dst2, b2):
    src = edge_index[0]
    dst = edge_index[1]
    eye8 = jnp.eye(HEADS, dtype=_f32)
    As1 = (a_src1[:, :, None] * eye8[:, None, :]).reshape(HEADS * HID, HEADS)
    Ad1 = (a_dst1[:, :, None] * eye8[:, None, :]).reshape(HEADS * HID, HEADS)

    ht0, ht1, ht2, ht3, P1, ploop1 = _stage1(x, W1, As1, Ad1)
    p1, s1p, acc1 = _sc1(src, dst, P1, ht0, ht1, ht2, ht3)
    h2T, P2 = _stage3(acc1, s1p, ploop1, ht0, ht1, ht2, ht3,
                      b1.reshape(1, -1), W2, a_src2, a_dst2)
    acc2 = _sc2(src, dst, P2, h2T)
    return _stage4(acc2, h2T, P2, b2.reshape(1, -1))


# speedup vs baseline: 54.0192x; 1.0357x over previous
import functools

import jax
import jax.numpy as jnp
from jax import lax
from jax.experimental import pallas as pl
from jax.experimental.pallas import tpu as pltpu
from jax.experimental.pallas import tpu_sc as plsc

N = 10000
E = 320000
D_IN = 128
HID = 64
HEADS = 8
D_OUT = 128

NC = 2
NS = 16
NW = NC * NS
EPW = E // NW
CH = 80
NCHUNK = EPW // CH
NPAD = 10240
RPT = NPAD // NS

_BLK = 2000

_f32 = jnp.float32
_i32 = jnp.int32


def _stage1_body(x_ref, w1_ref, as_ref, ad_ref,
                 ht0, ht1, ht2, ht3, p1tab, ploop):
    h = jnp.dot(x_ref[...], w1_ref[...], preferred_element_type=_f32)
    ht0[...] = h[:, 0:128]
    ht1[...] = h[:, 128:256]
    ht2[...] = h[:, 256:384]
    ht3[...] = h[:, 384:512]
    a_s = jnp.dot(h, as_ref[...], preferred_element_type=_f32)
    a_d = jnp.dot(h, ad_ref[...], preferred_element_type=_f32)
    p1tab[...] = jnp.concatenate([a_s, a_d], axis=1)
    el = a_s + a_d
    el = jnp.where(el >= 0, el, 0.2 * el)
    ploop[...] = jnp.exp(el)


def _stage1(x, W1, As1, Ad1):
    grid = (N // _BLK,)
    return pl.pallas_call(
        _stage1_body,
        grid=grid,
        in_specs=[
            pl.BlockSpec((_BLK, D_IN), lambda i: (i, 0)),
            pl.BlockSpec((D_IN, HEADS * HID), lambda i: (0, 0)),
            pl.BlockSpec((HEADS * HID, HEADS), lambda i: (0, 0)),
            pl.BlockSpec((HEADS * HID, HEADS), lambda i: (0, 0)),
        ],
        out_specs=[
            pl.BlockSpec((_BLK, 128), lambda i: (i, 0)),
            pl.BlockSpec((_BLK, 128), lambda i: (i, 0)),
            pl.BlockSpec((_BLK, 128), lambda i: (i, 0)),
            pl.BlockSpec((_BLK, 128), lambda i: (i, 0)),
            pl.BlockSpec((_BLK, 16), lambda i: (i, 0)),
            pl.BlockSpec((_BLK, 8), lambda i: (i, 0)),
        ],
        out_shape=[
            jax.ShapeDtypeStruct((N, 128), _f32),
            jax.ShapeDtypeStruct((N, 128), _f32),
            jax.ShapeDtypeStruct((N, 128), _f32),
            jax.ShapeDtypeStruct((N, 128), _f32),
            jax.ShapeDtypeStruct((N, 16), _f32),
            jax.ShapeDtypeStruct((N, 8), _f32),
        ],
    )(x, W1, As1, Ad1)


def _sc1_body(src_hbm, dst_hbm, p1tab_hbm, ht0, ht1, ht2, ht3,
              p1_out, s1_out, acc1_out,
              s_sh, acc_sh,
              src_v, dst_v, dsc_v, sa_rows, da_rows, pch, pt, rows,
              pav, pbv, zb16, zb128, sem_s, sem_d, sem_d2, sem_g, sem_z):
    cid = lax.axis_index("c")
    sid = lax.axis_index("s")
    wid = sid * NC + cid
    ebase = wid * EPW
    rbase = sid * RPT

    zv = jnp.zeros((16,), _f32)
    iota16 = lax.iota(_i32, 16)

    def _zb_init(i, c):
        for j in range(8):
            zb128[i, pl.ds(j * 16, 16)] = zv
        return c
    lax.fori_loop(0, 16, _zb_init, 0)

    def _zb16_init(i, c):
        zb16[i, :] = zv
        return c
    lax.fori_loop(0, 64, _zb16_init, 0)

    def _pt_init(i, c):
        pt[i, :] = zv
        return c
    lax.fori_loop(0, CH, _pt_init, 0)

    def _zero_acc():
        for k in range(40):
            pltpu.make_async_copy(zb128, acc_sh.at[pl.ds(rbase + k * 16, 16)],
                                  sem_z).start()
        for k in range(40):
            pltpu.make_async_copy(zb128, acc_sh.at[pl.ds(rbase + k * 16, 16)],
                                  sem_z).wait()

    for k in range(10):
        pltpu.make_async_copy(zb16, s_sh.at[pl.ds(rbase + k * 64, 64)],
                              sem_z).start()
    for k in range(40):
        pltpu.make_async_copy(zb128, acc_sh.at[pl.ds(rbase + k * 16, 16)],
                              sem_z).start()
    for k in range(10):
        pltpu.make_async_copy(zb16, s_sh.at[pl.ds(rbase + k * 64, 64)],
                              sem_z).wait()
    for k in range(40):
        pltpu.make_async_copy(zb128, acc_sh.at[pl.ds(rbase + k * 16, 16)],
                              sem_z).wait()
    plsc.subcore_barrier()

    def _issue_src(ci, b):
        pltpu.make_async_copy(src_hbm.at[pl.ds(ebase + ci * CH, CH)],
                              src_v[b], sem_s[b]).start()

    def _wait_src(ci, b):
        pltpu.make_async_copy(src_hbm.at[pl.ds(ebase + ci * CH, CH)],
                              src_v[b], sem_s[b]).wait()

    def _issue_dst(ci, b):
        pltpu.make_async_copy(dst_hbm.at[pl.ds(ebase + ci * CH, CH)],
                              dst_v[b], sem_d[b]).start()

    def _wait_dst(ci, b):
        pltpu.make_async_copy(dst_hbm.at[pl.ds(ebase + ci * CH, CH)],
                              dst_v[b], sem_d[b]).wait()

    def _issue_dsc(ci, b):
        pltpu.make_async_copy(dst_hbm.at[pl.ds(ebase + ci * CH, CH)],
                              dsc_v[b], sem_d2[b]).start()

    def _wait_dsc(ci, b):
        pltpu.make_async_copy(dst_hbm.at[pl.ds(ebase + ci * CH, CH)],
                              dsc_v[b], sem_d2[b]).wait()

    hts = [ht0, ht1, ht2, ht3]
    for pp in range(4):
        ht = hts[pp]

        def _issue_g(ci, b, _ht=ht, _pp=pp):
            pltpu.make_async_copy(_ht.at[src_v[b]], rows[b], sem_g[b]).start()
            if _pp == 0:
                pltpu.make_async_copy(p1tab_hbm.at[src_v[b]], sa_rows[b],
                                      sem_g[b]).start()
                pltpu.make_async_copy(p1tab_hbm.at[dst_v[b]], da_rows[b],
                                      sem_g[b]).start()
            else:
                pltpu.make_async_copy(
                    p1_out.at[2 * _pp, pl.ds(ebase + ci * CH, CH)],
                    pav[b], sem_g[b]).start()
                pltpu.make_async_copy(
                    p1_out.at[2 * _pp + 1, pl.ds(ebase + ci * CH, CH)],
                    pbv[b], sem_g[b]).start()

        def _wait_g(ci, b, _ht=ht, _pp=pp):
            pltpu.make_async_copy(_ht.at[src_v[b]], rows[b], sem_g[b]).wait()
            if _pp == 0:
                pltpu.make_async_copy(p1tab_hbm.at[src_v[b]], sa_rows[b],
                                      sem_g[b]).wait()
                pltpu.make_async_copy(p1tab_hbm.at[dst_v[b]], da_rows[b],
                                      sem_g[b]).wait()
            else:
                pltpu.make_async_copy(
                    p1_out.at[2 * _pp, pl.ds(ebase + ci * CH, CH)],
                    pav[b], sem_g[b]).wait()
                pltpu.make_async_copy(
                    p1_out.at[2 * _pp + 1, pl.ds(ebase + ci * CH, CH)],
                    pbv[b], sem_g[b]).wait()

        def _work(ci, b, _pp=pp):
            rb = rows[b]
            if _pp == 0:
                off = ebase + ci * CH
                for g in range(5):
                    ridx = iota16 + (g * 16)
                    for hh in range(8):
                        av = plsc.load_gather(
                            sa_rows[b], [ridx, jnp.full((16,), hh, _i32)])
                        bv = plsc.load_gather(
                            da_rows[b], [ridx, jnp.full((16,), 8 + hh, _i32)])
                        ev = av + bv
                        ev = jnp.where(ev >= 0, ev, 0.2 * ev)
                        pv = jnp.exp(ev)
                        pch[hh, pl.ds(g * 16, 16)] = pv
                        plsc.store_scatter(pt, [ridx, jnp.full((16,), hh, _i32)], pv)
                pltpu.make_async_copy(pch, p1_out.at[:, pl.ds(off, CH)],
                                      sem_z).start()
                _wait_dsc(ci, b)
                pltpu.sync_copy(pt, s_sh.at[dsc_v[b]], add=True)

                @plsc.parallel_loop(0, CH, 1, unroll=8)
                def _erow0(e):
                    eg = jnp.full((16,), e, _i32)
                    pa = plsc.load_gather(pch, [jnp.full((16,), 0, _i32), eg])
                    pb = plsc.load_gather(pch, [jnp.full((16,), 1, _i32), eg])
                    for j in range(4):
                        rb[e, pl.ds(j * 16, 16)] = rb[e, pl.ds(j * 16, 16)] * pa
                    for j in range(4, 8):
                        rb[e, pl.ds(j * 16, 16)] = rb[e, pl.ds(j * 16, 16)] * pb
                pltpu.make_async_copy(pch, p1_out.at[:, pl.ds(off, CH)],
                                      sem_z).wait()
            else:
                @plsc.parallel_loop(0, CH, 1, unroll=8)
                def _erow(e):
                    eg = jnp.full((16,), e, _i32)
                    pa = plsc.load_gather(pav[b], [eg])
                    pb = plsc.load_gather(pbv[b], [eg])
                    for j in range(4):
                        rb[e, pl.ds(j * 16, 16)] = rb[e, pl.ds(j * 16, 16)] * pa
                    for j in range(4, 8):
                        rb[e, pl.ds(j * 16, 16)] = rb[e, pl.ds(j * 16, 16)] * pb
                _wait_dsc(ci, b)
            pltpu.sync_copy(rb, acc_sh.at[dsc_v[b]], add=True)

        for b in (0, 1):
            _issue_src(b, b)
            _issue_dsc(b, b)
            if pp == 0:
                _issue_dst(b, b)
        _wait_src(0, 0)
        if pp == 0:
            _wait_dst(0, 0)
        _issue_g(0, 0)

        def _pair(k, c, _pp=pp, _issue_g=_issue_g, _wait_g=_wait_g, _work=_work):
            i = 2 * k + 1
            _wait_src(i, 1)
            if _pp == 0:
                _wait_dst(i, 1)
            _issue_g(i, 1)
            _wait_g(i - 1, 0)
            _issue_src(i + 1, 0)
            if _pp == 0:
                _issue_dst(i + 1, 0)
            _work(i - 1, 0)
            _issue_dsc(i + 1, 0)
            _wait_src(i + 1, 0)
            if _pp == 0:
                _wait_dst(i + 1, 0)
            _issue_g(i + 1, 0)
            _wait_g(i, 1)

            @pl.when(i + 2 < NCHUNK)
            def _():
                _issue_src(i + 2, 1)
                if _pp == 0:
                    _issue_dst(i + 2, 1)
            _work(i, 1)

            @pl.when(i + 2 < NCHUNK)
            def _():
                _issue_dsc(i + 2, 1)
            return c
        lax.fori_loop(0, (NCHUNK - 1) // 2, _pair, 0)
        _wait_g(NCHUNK - 1, 0)
        _work(NCHUNK - 1, 0)

        plsc.subcore_barrier()
        for k in range(5):
            pltpu.make_async_copy(acc_sh.at[pl.ds(rbase + k * 128, 128)],
                                  acc1_out.at[cid, pp, pl.ds(rbase + k * 128, 128)],
                                  sem_z).start()
        if pp == 0:
            for k in range(5):
                pltpu.make_async_copy(s_sh.at[pl.ds(rbase + k * 128, 128)],
                                      s1_out.at[cid, pl.ds(rbase + k * 128, 128)],
                                      sem_z).start()
        for k in range(5):
            pltpu.make_async_copy(acc_sh.at[pl.ds(rbase + k * 128, 128)],
                                  acc1_out.at[cid, pp, pl.ds(rbase + k * 128, 128)],
                                  sem_z).wait()
        if pp == 0:
            for k in range(5):
                pltpu.make_async_copy(s_sh.at[pl.ds(rbase + k * 128, 128)],
                                      s1_out.at[cid, pl.ds(rbase + k * 128, 128)],
                                      sem_z).wait()
        if pp < 3:
            _zero_acc()
            plsc.subcore_barrier()


def _sc1(src, dst, P1, ht0, ht1, ht2, ht3):
    mesh = plsc.VectorSubcoreMesh(core_axis_name="c", subcore_axis_name="s",
                                  num_cores=NC, num_subcores=NS)
    f = functools.partial(
        pl.kernel,
        out_type=[
            jax.ShapeDtypeStruct((8, E), _f32),
            jax.ShapeDtypeStruct((NC, NPAD, 16), _f32),
            jax.ShapeDtypeStruct((NC, 4, NPAD, 128), _f32),
        ],
        mesh=mesh,
        compiler_params=pltpu.CompilerParams(use_tc_tiling_on_sc=False, needs_layout_passes=False),
        scratch_types=[
            pltpu.VMEM_SHARED((NPAD, 16), _f32),
            pltpu.VMEM_SHARED((NPAD, 128), _f32),
            [pltpu.VMEM((CH,), _i32), pltpu.VMEM((CH,), _i32)],
            [pltpu.VMEM((CH,), _i32), pltpu.VMEM((CH,), _i32)],
            [pltpu.VMEM((CH,), _i32), pltpu.VMEM((CH,), _i32)],
            [pltpu.VMEM((CH, 16), _f32), pltpu.VMEM((CH, 16), _f32)],
            [pltpu.VMEM((CH, 16), _f32), pltpu.VMEM((CH, 16), _f32)],
            pltpu.VMEM((8, CH), _f32),
            pltpu.VMEM((CH, 16), _f32),
            [pltpu.VMEM((CH, 128), _f32), pltpu.VMEM((CH, 128), _f32)],
            [pltpu.VMEM((CH,), _f32), pltpu.VMEM((CH,), _f32)],
            [pltpu.VMEM((CH,), _f32), pltpu.VMEM((CH,), _f32)],
            pltpu.VMEM((64, 16), _f32),
            pltpu.VMEM((16, 128), _f32),
            [pltpu.SemaphoreType.DMA, pltpu.SemaphoreType.DMA],
            [pltpu.SemaphoreType.DMA, pltpu.SemaphoreType.DMA],
            [pltpu.SemaphoreType.DMA, pltpu.SemaphoreType.DMA],
            [pltpu.SemaphoreType.DMA, pltpu.SemaphoreType.DMA],
            pltpu.SemaphoreType.DMA,
        ],
    )(_sc1_body)
    return f(src, dst, P1, ht0, ht1, ht2, ht3)


def _stage3_body(acc1_ref, s1p_ref, ploop_ref, ht0, ht1, ht2, ht3,
                 b1_ref, w2_ref, as2_ref, ad2_ref,
                 h2t_ref, p2tab_ref):
    blk = ploop_ref.shape[0]
    s_tot = (s1p_ref[0, :, 0:8] + s1p_ref[1, :, 0:8] + ploop_ref[...])
    inv = 1.0 / (s_tot + 1e-16)
    hts = [ht0, ht1, ht2, ht3]
    cols = []
    for pp in range(4):
        acc = acc1_ref[0, pp] + acc1_ref[1, pp]
        hta = hts[pp][...]
        pw = jnp.concatenate(
            [jnp.broadcast_to(ploop_ref[:, 2 * pp:2 * pp + 1], (blk, 64)),
             jnp.broadcast_to(ploop_ref[:, 2 * pp + 1:2 * pp + 2], (blk, 64))],
            axis=1)
        iw = jnp.concatenate(
            [jnp.broadcast_to(inv[:, 2 * pp:2 * pp + 1], (blk, 64)),
             jnp.broadcast_to(inv[:, 2 * pp + 1:2 * pp + 2], (blk, 64))],
            axis=1)
        num = acc + pw * hta
        cols.append(num * iw)
    h1 = jnp.concatenate(cols, axis=1) + b1_ref[...]
    h1 = jnp.maximum(h1, 0.0)
    h2 = jnp.dot(h1, w2_ref[...], preferred_element_type=_f32)
    a_s2 = jnp.sum(h2 * as2_ref[...], axis=1, keepdims=True)
    a_d2 = jnp.sum(h2 * ad2_ref[...], axis=1, keepdims=True)
    el = a_s2 + a_d2
    el = jnp.where(el >= 0, el, 0.2 * el)
    pl2 = jnp.exp(el)
    h2t_ref[...] = jnp.concatenate(
        [h2, jnp.ones((blk, 1), _f32), jnp.zeros((blk, 15), _f32)], axis=1)
    p2tab_ref[...] = jnp.concatenate(
        [a_s2, a_d2, pl2, jnp.zeros((blk, 13), _f32)], axis=1)


def _stage3(acc1, s1p, ploop1, ht0, ht1, ht2, ht3, b1, W2, a_src2, a_dst2):
    grid = (N // _BLK,)
    return pl.pallas_call(
        _stage3_body,
        grid=grid,
        in_specs=[
            pl.BlockSpec((NC, 4, _BLK, 128), lambda i: (0, 0, i, 0)),
            pl.BlockSpec((NC, _BLK, 16), lambda i: (0, i, 0)),
            pl.BlockSpec((_BLK, 8), lambda i: (i, 0)),
            pl.BlockSpec((_BLK, 128), lambda i: (i, 0)),
            pl.BlockSpec((_BLK, 128), lambda i: (i, 0)),
            pl.BlockSpec((_BLK, 128), lambda i: (i, 0)),
            pl.BlockSpec((_BLK, 128), lambda i: (i, 0)),
            pl.BlockSpec((1, 512), lambda i: (0, 0)),
            pl.BlockSpec((512, 128), lambda i: (0, 0)),
            pl.BlockSpec((1, 128), lambda i: (0, 0)),
            pl.BlockSpec((1, 128), lambda i: (0, 0)),
        ],
        out_specs=[
            pl.BlockSpec((_BLK, 144), lambda i: (i, 0)),
            pl.BlockSpec((_BLK, 16), lambda i: (i, 0)),
        ],
        out_shape=[
            jax.ShapeDtypeStruct((N, 144), _f32),
            jax.ShapeDtypeStruct((N, 16), _f32),
        ],
    )(acc1, s1p, ploop1, ht0, ht1, ht2, ht3, b1, W2, a_src2, a_dst2)


def _sc2_body(src_hbm, dst_hbm, p2tab_hbm, h2t_hbm,
              acc2_out,
              acc_sh,
              src_v, dst_v, dsc_v, sa_rows, da_rows, p2ch, rows,
              zb144, sem_s, sem_d, sem_d2, sem_g, sem_z):
    cid = lax.axis_index("c")
    sid = lax.axis_index("s")
    wid = sid * NC + cid
    ebase = wid * EPW
    rbase = sid * RPT

    zv = jnp.zeros((16,), _f32)
    iota16 = lax.iota(_i32, 16)

    def _zb_init(i, c):
        for j in range(9):
            zb144[i, pl.ds(j * 16, 16)] = zv
        return c
    lax.fori_loop(0, 16, _zb_init, 0)

    for k in range(40):
        pltpu.make_async_copy(zb144, acc_sh.at[pl.ds(rbase + k * 16, 16)],
                              sem_z).start()
    for k in range(40):
        pltpu.make_async_copy(zb144, acc_sh.at[pl.ds(rbase + k * 16, 16)],
                              sem_z).wait()
    plsc.subcore_barrier()

    def _issue_src(ci, b):
        pltpu.make_async_copy(src_hbm.at[pl.ds(ebase + ci * CH, CH)],
                              src_v[b], sem_s[b]).start()

    def _wait_src(ci, b):
        pltpu.make_async_copy(src_hbm.at[pl.ds(ebase + ci * CH, CH)],
                              src_v[b], sem_s[b]).wait()

    def _issue_dst(ci, b):
        pltpu.make_async_copy(dst_hbm.at[pl.ds(ebase + ci * CH, CH)],
                              dst_v[b], sem_d[b]).start()

    def _wait_dst(ci, b):
        pltpu.make_async_copy(dst_hbm.at[pl.ds(ebase + ci * CH, CH)],
                              dst_v[b], sem_d[b]).wait()

    def _issue_dsc(ci, b):
        pltpu.make_async_copy(dst_hbm.at[pl.ds(ebase + ci * CH, CH)],
                              dsc_v[b], sem_d2[b]).start()

    def _wait_dsc(ci, b):
        pltpu.make_async_copy(dst_hbm.at[pl.ds(ebase + ci * CH, CH)],
                              dsc_v[b], sem_d2[b]).wait()

    def _issue_g(ci, b):
        pltpu.make_async_copy(p2tab_hbm.at[src_v[b]], sa_rows[b],
                              sem_g[b]).start()
        pltpu.make_async_copy(p2tab_hbm.at[dst_v[b]], da_rows[b],
                              sem_g[b]).start()
        pltpu.make_async_copy(h2t_hbm.at[src_v[b]], rows[b],
                              sem_g[b]).start()

    def _wait_g(ci, b):
        pltpu.make_async_copy(p2tab_hbm.at[src_v[b]], sa_rows[b],
                              sem_g[b]).wait()
        pltpu.make_async_copy(p2tab_hbm.at[dst_v[b]], da_rows[b],
                              sem_g[b]).wait()
        pltpu.make_async_copy(h2t_hbm.at[src_v[b]], rows[b],
                              sem_g[b]).wait()

    def _work(ci, b):
        rb = rows[b]
        for g in range(5):
            ridx = iota16 + (g * 16)
            av = plsc.load_gather(sa_rows[b], [ridx, jnp.full((16,), 0, _i32)])
            bv = plsc.load_gather(da_rows[b], [ridx, jnp.full((16,), 1, _i32)])
            ev = av + bv
            ev = jnp.where(ev >= 0, ev, 0.2 * ev)
            p2ch[pl.ds(g * 16, 16)] = jnp.exp(ev)

        @plsc.parallel_loop(0, CH, 1, unroll=8)
        def _erow(e):
            pv = plsc.load_gather(p2ch, [jnp.full((16,), e, _i32)])
            for j in range(9):
                rb[e, pl.ds(j * 16, 16)] = rb[e, pl.ds(j * 16, 16)] * pv
        _wait_dsc(ci, b)
        pltpu.sync_copy(rb, acc_sh.at[dsc_v[b]], add=True)

    for b in (0, 1):
        _issue_src(b, b)
        _issue_dst(b, b)
        _issue_dsc(b, b)
    _wait_src(0, 0)
    _wait_dst(0, 0)
    _issue_g(0, 0)

    def _pair(k, c):
        i = 2 * k + 1
        _wait_src(i, 1)
        _wait_dst(i, 1)
        _issue_g(i, 1)
        _wait_g(i - 1, 0)
        _issue_src(i + 1, 0)
        _issue_dst(i + 1, 0)
        _work(i - 1, 0)
        _issue_dsc(i + 1, 0)
        _wait_src(i + 1, 0)
        _wait_dst(i + 1, 0)
        _issue_g(i + 1, 0)
        _wait_g(i, 1)

        @pl.when(i + 2 < NCHUNK)
        def _():
            _issue_src(i + 2, 1)
            _issue_dst(i + 2, 1)
        _work(i, 1)

        @pl.when(i + 2 < NCHUNK)
        def _():
            _issue_dsc(i + 2, 1)
        return c
    lax.fori_loop(0, (NCHUNK - 1) // 2, _pair, 0)
    _wait_g(NCHUNK - 1, 0)
    _work(NCHUNK - 1, 0)

    plsc.subcore_barrier()
    for k in range(5):
        pltpu.make_async_copy(acc_sh.at[pl.ds(rbase + k * 128, 128)],
                              acc2_out.at[cid, pl.ds(rbase + k * 128, 128)],
                              sem_z).start()
    for k in range(5):
        pltpu.make_async_copy(acc_sh.at[pl.ds(rbase + k * 128, 128)],
                              acc2_out.at[cid, pl.ds(rbase + k * 128, 128)],
                              sem_z).wait()


def _sc2(src, dst, P2, h2T):
    mesh = plsc.VectorSubcoreMesh(core_axis_name="c", subcore_axis_name="s",
                                  num_cores=NC, num_subcores=NS)
    f = functools.partial(
        pl.kernel,
        out_type=jax.ShapeDtypeStruct((NC, NPAD, 144), _f32),
        mesh=mesh,
        compiler_params=pltpu.CompilerParams(use_tc_tiling_on_sc=False, needs_layout_passes=False),
        scratch_types=[
            pltpu.VMEM_SHARED((NPAD, 144), _f32),
            [pltpu.VMEM((CH,), _i32), pltpu.VMEM((CH,), _i32)],
            [pltpu.VMEM((CH,), _i32), pltpu.VMEM((CH,), _i32)],
            [pltpu.VMEM((CH,), _i32), pltpu.VMEM((CH,), _i32)],
            [pltpu.VMEM((CH, 16), _f32), pltpu.VMEM((CH, 16), _f32)],
            [pltpu.VMEM((CH, 16), _f32), pltpu.VMEM((CH, 16), _f32)],
            pltpu.VMEM((CH,), _f32),
            [pltpu.VMEM((CH, 144), _f32), pltpu.VMEM((CH, 144), _f32)],
            pltpu.VMEM((16, 144), _f32),
            [pltpu.SemaphoreType.DMA, pltpu.SemaphoreType.DMA],
            [pltpu.SemaphoreType.DMA, pltpu.SemaphoreType.DMA],
            [pltpu.SemaphoreType.DMA, pltpu.SemaphoreType.DMA],
            [pltpu.SemaphoreType.DMA, pltpu.SemaphoreType.DMA],
            pltpu.SemaphoreType.DMA,
        ],
    )(_sc2_body)
    return f(src, dst, P2, h2T)


def _stage4_body(acc2_ref, h2t_ref, p2tab_ref, b2_ref, out_ref):
    accs = acc2_ref[0] + acc2_ref[1]
    pl2 = p2tab_ref[:, 2:3]
    num = accs[:, 0:128] + pl2 * h2t_ref[:, 0:128]
    s = accs[:, 128:129] + pl2
    o = num / (s + 1e-16) + b2_ref[...]
    m = jnp.max(o, axis=1, keepdims=True)
    z = o - m
    out_ref[...] = z - jnp.log(jnp.sum(jnp.exp(z), axis=1, keepdims=True))


def _stage4(acc2, h2T, P2, b2):
    grid = (N // _BLK,)
    return pl.pallas_call(
        _stage4_body,
        grid=grid,
        in_specs=[
            pl.BlockSpec((NC, _BLK, 144), lambda i: (0, i, 0)),
            pl.BlockSpec((_BLK, 144), lambda i: (i, 0)),
            pl.BlockSpec((_BLK, 16), lambda i: (i, 0)),
            pl.BlockSpec((1, 128), lambda i: (0, 0)),
        ],
        out_specs=pl.BlockSpec((_BLK, 128), lambda i: (i, 0)),
        out_shape=jax.ShapeDtypeStruct((N, D_OUT), _f32),
    )(acc2, h2T, P2, b2)


def kernel(x, edge_index, W1, a_src1, a_dst1, b1, W2, a_src2, a_dst2, b2):
    src = edge_index[0]
    dst = edge_index[1]
    eye8 = jnp.eye(HEADS, dtype=_f32)
    As1 = (a_src1[:, :, None] * eye8[:, None, :]).reshape(HEADS * HID, HEADS)
    Ad1 = (a_dst1[:, :, None] * eye8[:, None, :]).reshape(HEADS * HID, HEADS)

    ht0, ht1, ht2, ht3, P1, ploop1 = _stage1(x, W1, As1, Ad1)
    p1, s1p, acc1 = _sc1(src, dst, P1, ht0, ht1, ht2, ht3)
    h2T, P2 = _stage3(acc1, s1p, ploop1, ht0, ht1, ht2, ht3,
                      b1.reshape(1, -1), W2, a_src2, a_dst2)
    acc2 = _sc2(src, dst, P2, h2T)
    return _stage4(acc2, h2T, P2, b2.reshape(1, -1))
